# baseline (speedup 1.0000x reference)
import functools

import jax
import jax.numpy as jnp
from jax import lax
from jax.experimental import pallas as pl
from jax.experimental.pallas import tpu as pltpu
from jax.experimental.pallas import tpu_sc as plsc

N = 10000
D = 128
NC = 2
NS = 16
NW = NC * NS
CH = 64

ACC_ROWS = 10112
SEG = ACC_ROWS // NS
COUNT_PAD = 10240
CSEG = COUNT_PAD // NS


def _pad_edges(src, dst):
    e = src.shape[0]
    per = NW * CH
    j = -(-e // per)
    j = -(-j // 4) * 4
    pad = j * per - e
    srcp = jnp.concatenate([src, jnp.zeros((pad,), jnp.int32)])
    dstp = jnp.concatenate([dst, jnp.full((pad,), N, jnp.int32)])
    return jnp.stack([srcp.reshape(NW, j, CH), dstp.reshape(NW, j, CH)], axis=2)


def _mesh():
    return plsc.VectorSubcoreMesh(core_axis_name="c", subcore_axis_name="s")


def _sc_degree(edges):
    nj = edges.shape[1]

    @functools.partial(
        pl.kernel,
        mesh=_mesh(),
        out_type=jax.ShapeDtypeStruct((NC, COUNT_PAD), jnp.float32),
        scratch_types=[
            pltpu.VMEM((CH,), jnp.float32),
            pltpu.VMEM((CH,), jnp.int32),
            pltpu.VMEM((CSEG,), jnp.float32),
            pltpu.VMEM_SHARED((COUNT_PAD,), jnp.float32),
        ],
    )
    def deg_k(edges_hbm, out_hbm, ones_v, idx_v, zrow_v, counts):
        cid = lax.axis_index("c")
        sid = lax.axis_index("s")
        wid = sid * NC + cid
        z16 = jnp.zeros((16,), jnp.float32)
        o16 = jnp.ones((16,), jnp.float32)
        for k in range(CH // 16):
            ones_v[pl.ds(k * 16, 16)] = o16

        def zb(i, c):
            zrow_v[pl.ds(i * 16, 16)] = z16
            return c

        lax.fori_loop(0, CSEG // 16, zb, 0)
        pltpu.sync_copy(zrow_v, counts.at[pl.ds(sid * CSEG, CSEG)])
        plsc.subcore_barrier()

        def body(j, c):
            pltpu.sync_copy(edges_hbm.at[wid, j, 1], idx_v)
            pltpu.sync_copy(ones_v, counts.at[idx_v], add=True)
            return c

        lax.fori_loop(0, nj, body, 0)
        plsc.subcore_barrier()
        pltpu.sync_copy(counts.at[pl.ds(sid * CSEG, CSEG)],
                        out_hbm.at[cid, pl.ds(sid * CSEG, CSEG)])

    return deg_k(edges)


_NB = 4


def _sc_scatter(table, edges):
    nj = edges.shape[1]
    assert nj % _NB == 0 and nj >= 2 * _NB

    @functools.partial(
        pl.kernel,
        mesh=_mesh(),
        out_type=jax.ShapeDtypeStruct((NC, ACC_ROWS, D), jnp.float32),
        scratch_types=(
            [pltpu.VMEM((2, CH), jnp.int32) for _ in range(_NB)]
            + [pltpu.VMEM((CH, D), jnp.float32) for _ in range(_NB)]
            + [pltpu.SemaphoreType.DMA for _ in range(3 * _NB)]
            + [pltpu.VMEM_SHARED((ACC_ROWS, D), jnp.float32)]
        ),
    )
    def scat_k(tab_hbm, edges_hbm, out_hbm, *refs):
        idx = refs[0:_NB]
        rows = refs[_NB:2 * _NB]
        isem = refs[2 * _NB:3 * _NB]
        gsem = refs[3 * _NB:4 * _NB]
        ssem = refs[4 * _NB:5 * _NB]
        acc = refs[5 * _NB]
        cid = lax.axis_index("c")
        sid = lax.axis_index("s")
        wid = sid * NC + cid
        z16 = jnp.zeros((16,), jnp.float32)

        def zrow(r, c):
            for k in range(D // 16):
                rows[0][r, pl.ds(k * 16, 16)] = z16
            return c

        lax.fori_loop(0, CH, zrow, 0)
        base = sid * SEG
        nfull, rem = SEG // CH, SEG % CH
        for c in range(nfull):
            pltpu.sync_copy(rows[0], acc.at[pl.ds(base + c * CH, CH)])
        if rem:
            pltpu.sync_copy(rows[0].at[pl.ds(0, rem)],
                            acc.at[pl.ds(base + nfull * CH, rem)])
        plsc.subcore_barrier()

        def idx_start(s, j):
            pltpu.async_copy(edges_hbm.at[wid, j], idx[s], isem[s])

        def idx_wait(s, j):
            pltpu.make_async_copy(edges_hbm.at[wid, j], idx[s], isem[s]).wait()

        def gat_start(s):
            pltpu.async_copy(tab_hbm.at[idx[s].at[0]], rows[s], gsem[s])

        def gat_wait(s):
            pltpu.make_async_copy(tab_hbm.at[idx[s].at[0]], rows[s],
                                  gsem[s]).wait()

        def sca_start(s):
            pltpu.async_copy(rows[s], acc.at[idx[s].at[1]], ssem[s], add=True)

        def sca_wait(s):
            pltpu.make_async_copy(rows[s], acc.at[idx[s].at[1]],
                                  ssem[s]).wait()

        idx_start(0, 0)
        for v in range(3):
            idx_wait(v, v)
            gat_start(v)
            idx_start(v + 1, v + 1)
            if v >= 2:
                gat_wait((v + 2) % _NB)
                sca_start((v + 2) % _NB)

        def steady(i, c):
            v0 = 3 + i * _NB
            for r in range(_NB):
                b = (3 + r) % _NB
                v = v0 + r
                idx_wait(b, v)
                gat_start(b)
                sca_wait((b + 1) % _NB)
                idx_start((b + 1) % _NB, v + 1)
                gat_wait((b + 2) % _NB)
                sca_start((b + 2) % _NB)
            return c

        lax.fori_loop(0, (nj - 2 - 3 + 1) // _NB, steady, 0)

        bl = (nj - 1) % _NB
        idx_wait(bl, nj - 1)
        gat_start(bl)
        for v in (nj - 3, nj - 2, nj - 1):
            s = v % _NB
            gat_wait(s)
            sca_start(s)
        for s in range(_NB):
            sca_wait(s)

        plsc.subcore_barrier()
        pltpu.sync_copy(acc.at[pl.ds(base, SEG)],
                        out_hbm.at[cid, pl.ds(base, SEG)])

    return scat_k(table, edges)


_BR = 1000


def _tc_scale_matmul(x, c0, c1, w1):
    def body(x_ref, c0_ref, c1_ref, w_ref, g_ref, dinv_ref):
        dinv = lax.rsqrt(c0_ref[...] + c1_ref[...] + 1.0)
        g_ref[...] = jnp.dot(x_ref[...] * dinv, w_ref[...],
                             preferred_element_type=jnp.float32)
        dinv_ref[...] = dinv

    n = x.shape[0]
    return pl.pallas_call(
        body,
        grid=(n // _BR,),
        in_specs=[
            pl.BlockSpec((_BR, D), lambda i: (i, 0)),
            pl.BlockSpec((_BR, 1), lambda i: (i, 0)),
            pl.BlockSpec((_BR, 1), lambda i: (i, 0)),
            pl.BlockSpec((D, D), lambda i: (0, 0)),
        ],
        out_specs=[
            pl.BlockSpec((_BR, D), lambda i: (i, 0)),
            pl.BlockSpec((_BR, 1), lambda i: (i, 0)),
        ],
        out_shape=[
            jax.ShapeDtypeStruct((n, D), jnp.float32),
            jax.ShapeDtypeStruct((n, 1), jnp.float32),
        ],
    )(x, c0, c1, w1)


def _tc_mid(p0, p1, g1, dinv, b1, w2):
    def body(p0_ref, p1_ref, g_ref, dinv_ref, b_ref, w_ref, o_ref):
        t = dinv_ref[...] * (p0_ref[...] + p1_ref[...] + g_ref[...]) + b_ref[...]
        r = jnp.maximum(t, 0.0)
        o_ref[...] = dinv_ref[...] * jnp.dot(r, w_ref[...],
                                             preferred_element_type=jnp.float32)

    n = g1.shape[0]
    return pl.pallas_call(
        body,
        grid=(n // _BR,),
        in_specs=[
            pl.BlockSpec((_BR, D), lambda i: (i, 0)),
            pl.BlockSpec((_BR, D), lambda i: (i, 0)),
            pl.BlockSpec((_BR, D), lambda i: (i, 0)),
            pl.BlockSpec((_BR, 1), lambda i: (i, 0)),
            pl.BlockSpec((1, D), lambda i: (0, 0)),
            pl.BlockSpec((D, D), lambda i: (0, 0)),
        ],
        out_specs=pl.BlockSpec((_BR, D), lambda i: (i, 0)),
        out_shape=jax.ShapeDtypeStruct((n, D), jnp.float32),
    )(p0, p1, g1, dinv, b1, w2)


def _tc_final(q0, q1, g2, dinv, b2):
    def body(q0_ref, q1_ref, g_ref, dinv_ref, b_ref, o_ref):
        o_ref[...] = dinv_ref[...] * (
            q0_ref[...] + q1_ref[...] + g_ref[...]) + b_ref[...]

    n = g2.shape[0]
    return pl.pallas_call(
        body,
        grid=(n // _BR,),
        in_specs=[
            pl.BlockSpec((_BR, D), lambda i: (i, 0)),
            pl.BlockSpec((_BR, D), lambda i: (i, 0)),
            pl.BlockSpec((_BR, D), lambda i: (i, 0)),
            pl.BlockSpec((_BR, 1), lambda i: (i, 0)),
            pl.BlockSpec((1, D), lambda i: (0, 0)),
        ],
        out_specs=pl.BlockSpec((_BR, D), lambda i: (i, 0)),
        out_shape=jax.ShapeDtypeStruct((n, D), jnp.float32),
    )(q0, q1, g2, dinv, b2)


def kernel(x, edge_index, W1, b1, W2, b2):
    ei = edge_index.astype(jnp.int32)
    edges = _pad_edges(ei[0], ei[1])

    counts = _sc_degree(edges)
    c0 = counts[0, :N].reshape(N, 1)
    c1 = counts[1, :N].reshape(N, 1)

    g1, dinv = _tc_scale_matmul(x, c0, c1, W1)
    p = _sc_scatter(g1, edges)
    g2 = _tc_mid(p[0, :N], p[1, :N], g1, dinv, b1.reshape(1, D), W2)
    q = _sc_scatter(g2, edges)
    return _tc_final(q[0, :N], q[1, :N], g2, dinv, b2.reshape(1, D))


# speedup vs baseline: 1.9773x; 1.9773x over previous
import functools

import jax
import jax.numpy as jnp
from jax import lax
from jax.experimental import pallas as pl
from jax.experimental.pallas import tpu as pltpu
from jax.experimental.pallas import tpu_sc as plsc

N = 10000
D = 128
NC = 2
NS = 16
NW = NC * NS
CH = 64
F0 = 0.8

ACC_ROWS = 10112
SEG = ACC_ROWS // NS
COUNT_PAD = 10240
CSEG = COUNT_PAD // NS

_NB = 4


def _edge_layout(e):
    chunks = -(-e // CH)
    j0 = max(-(-int(chunks * F0) // NS), 2 * _NB)
    j0 = -(-j0 // _NB) * _NB
    j1 = max(-(-max(chunks - NS * j0, 0) // NS), 2 * _NB)
    j1 = -(-j1 // _NB) * _NB
    return j0, j1


def _pad_edges(src, dst, j0, j1):
    e = src.shape[0]
    totc = NS * (j0 + j1)
    pad = totc * CH - e
    srcp = jnp.concatenate([src, jnp.zeros((pad,), jnp.int32)]).reshape(totc, CH)
    dstp = jnp.concatenate([dst, jnp.full((pad,), N, jnp.int32)]).reshape(totc, CH)
    return srcp, dstp


def _mesh():
    return plsc.VectorSubcoreMesh(core_axis_name="c", subcore_axis_name="s")


def _sc_degree(dstp):
    njd = dstp.shape[0] // NW
    assert njd % 2 == 0 and njd >= 2

    @functools.partial(
        pl.kernel,
        mesh=_mesh(),
        out_type=jax.ShapeDtypeStruct((NC, COUNT_PAD), jnp.float32),
        scratch_types=[
            pltpu.VMEM((CH,), jnp.float32),
            pltpu.VMEM((CH,), jnp.int32),
            pltpu.VMEM((CH,), jnp.int32),
            pltpu.SemaphoreType.DMA,
            pltpu.SemaphoreType.DMA,
            pltpu.VMEM((CSEG,), jnp.float32),
            pltpu.VMEM_SHARED((COUNT_PAD,), jnp.float32),
        ],
    )
    def deg_k(dst_hbm, out_hbm, ones_v, i0, i1, s0, s1, zrow_v, counts):
        cid = lax.axis_index("c")
        sid = lax.axis_index("s")
        base = (sid * NC + cid) * njd
        idx, sem = (i0, i1), (s0, s1)
        z16 = jnp.zeros((16,), jnp.float32)
        o16 = jnp.ones((16,), jnp.float32)
        for k in range(CH // 16):
            ones_v[pl.ds(k * 16, 16)] = o16

        def zb(i, c):
            zrow_v[pl.ds(i * 16, 16)] = z16
            return c

        lax.fori_loop(0, CSEG // 16, zb, 0)
        pltpu.sync_copy(zrow_v, counts.at[pl.ds(sid * CSEG, CSEG)])
        plsc.subcore_barrier()

        def idx_start(b, j):
            pltpu.async_copy(dst_hbm.at[base + j], idx[b], sem[b])

        def visit(b, j, prefetch):
            pltpu.make_async_copy(dst_hbm.at[base + j], idx[b], sem[b]).wait()
            if prefetch:
                idx_start(1 - b, j + 1)
            pltpu.sync_copy(ones_v, counts.at[idx[b]], add=True)

        idx_start(0, 0)
        visit(0, 0, True)

        def steady(i, c):
            v = 1 + 2 * i
            visit(1, v, True)
            visit(0, v + 1, True)
            return c

        lax.fori_loop(0, (njd - 2) // 2, steady, 0)
        visit(1, njd - 1, False)

        plsc.subcore_barrier()
        pltpu.sync_copy(counts.at[pl.ds(sid * CSEG, CSEG)],
                        out_hbm.at[cid, pl.ds(sid * CSEG, CSEG)])

    return deg_k(dstp)


def _sc_scatter(table, srcp, dstp, j0, j1):
    for nj in (j0, j1):
        assert nj % _NB == 0 and nj >= 2 * _NB

    @functools.partial(
        pl.kernel,
        mesh=_mesh(),
        out_type=jax.ShapeDtypeStruct((NC, ACC_ROWS, D), jnp.float32),
        scratch_types=(
            [pltpu.VMEM((CH,), jnp.int32) for _ in range(2 * _NB)]
            + [pltpu.VMEM((CH, D), jnp.float32) for _ in range(_NB)]
            + [pltpu.SemaphoreType.DMA for _ in range(3 * _NB)]
            + [pltpu.VMEM_SHARED((ACC_ROWS, D), jnp.float32)]
        ),
    )
    def scat_k(tab_hbm, src_hbm, dst_hbm, out_hbm, *refs):
        sidx = refs[0:_NB]
        didx = refs[_NB:2 * _NB]
        rows = refs[2 * _NB:3 * _NB]
        isem = refs[3 * _NB:4 * _NB]
        gsem = refs[4 * _NB:5 * _NB]
        ssem = refs[5 * _NB:6 * _NB]
        acc = refs[6 * _NB]
        cid = lax.axis_index("c")
        sid = lax.axis_index("s")
        z16 = jnp.zeros((16,), jnp.float32)

        def zrow(r, c):
            for k in range(D // 16):
                rows[0][r, pl.ds(k * 16, 16)] = z16
            return c

        lax.fori_loop(0, CH, zrow, 0)
        base_r = sid * SEG
        nfull, rem = SEG // CH, SEG % CH
        for c in range(nfull):
            pltpu.sync_copy(rows[0], acc.at[pl.ds(base_r + c * CH, CH)])
        if rem:
            pltpu.sync_copy(rows[0].at[pl.ds(0, rem)],
                            acc.at[pl.ds(base_r + nfull * CH, rem)])
        plsc.subcore_barrier()

        def pipe(nj, base):
            def idx_start(s, j):
                pltpu.async_copy(src_hbm.at[base + j], sidx[s], isem[s])
                pltpu.async_copy(dst_hbm.at[base + j], didx[s], isem[s])

            def idx_wait(s, j):
                pltpu.make_async_copy(src_hbm.at[base + j], sidx[s],
                                      isem[s]).wait()
                pltpu.make_async_copy(dst_hbm.at[base + j], didx[s],
                                      isem[s]).wait()

            def gat_start(s):
                pltpu.async_copy(tab_hbm.at[sidx[s]], rows[s], gsem[s])

            def gat_wait(s):
                pltpu.make_async_copy(tab_hbm.at[sidx[s]], rows[s],
                                      gsem[s]).wait()

            def sca_start(s):
                pltpu.async_copy(rows[s], acc.at[didx[s]], ssem[s], add=True)

            def sca_wait(s):
                pltpu.make_async_copy(rows[s], acc.at[didx[s]],
                                      ssem[s]).wait()

            idx_start(0, 0)
            for v in range(3):
                idx_wait(v, v)
                gat_start(v)
                idx_start(v + 1, v + 1)
                if v >= 2:
                    gat_wait((v + 2) % _NB)
                    sca_start((v + 2) % _NB)

            def steady(i, c):
                v0 = 3 + i * _NB
                for r in range(_NB):
                    b = (3 + r) % _NB
                    v = v0 + r
                    idx_wait(b, v)
                    gat_start(b)
                    sca_wait((b + 1) % _NB)
                    idx_start((b + 1) % _NB, v + 1)
                    gat_wait((b + 2) % _NB)
                    sca_start((b + 2) % _NB)
                return c

            lax.fori_loop(0, (nj - 2 - 3 + 1) // _NB, steady, 0)

            bl = (nj - 1) % _NB
            idx_wait(bl, nj - 1)
            gat_start(bl)
            for v in (nj - 3, nj - 2, nj - 1):
                s = v % _NB
                gat_wait(s)
                sca_start(s)
            for s in range(_NB):
                sca_wait(s)

        @pl.when(cid == 0)
        def _():
            pipe(j0, sid * j0)

        @pl.when(cid == 1)
        def _():
            pipe(j1, NS * j0 + sid * j1)

        plsc.subcore_barrier()
        pltpu.sync_copy(acc.at[pl.ds(base_r, SEG)],
                        out_hbm.at[cid, pl.ds(base_r, SEG)])

    return scat_k(table, srcp, dstp)


_BR = 1000


def _tc_scale_matmul(x, c0, c1, w1):
    def body(x_ref, c0_ref, c1_ref, w_ref, g_ref, dinv_ref):
        dinv = lax.rsqrt(c0_ref[...] + c1_ref[...] + 1.0)
        g_ref[...] = jnp.dot(x_ref[...] * dinv, w_ref[...],
                             preferred_element_type=jnp.float32)
        dinv_ref[...] = dinv

    n = x.shape[0]
    return pl.pallas_call(
        body,
        grid=(n // _BR,),
        in_specs=[
            pl.BlockSpec((_BR, D), lambda i: (i, 0)),
            pl.BlockSpec((_BR, 1), lambda i: (i, 0)),
            pl.BlockSpec((_BR, 1), lambda i: (i, 0)),
            pl.BlockSpec((D, D), lambda i: (0, 0)),
        ],
        out_specs=[
            pl.BlockSpec((_BR, D), lambda i: (i, 0)),
            pl.BlockSpec((_BR, 1), lambda i: (i, 0)),
        ],
        out_shape=[
            jax.ShapeDtypeStruct((n, D), jnp.float32),
            jax.ShapeDtypeStruct((n, 1), jnp.float32),
        ],
    )(x, c0, c1, w1)


def _tc_mid(p0, p1, g1, dinv, b1, w2):
    def body(p0_ref, p1_ref, g_ref, dinv_ref, b_ref, w_ref, o_ref):
        t = dinv_ref[...] * (p0_ref[...] + p1_ref[...] + g_ref[...]) + b_ref[...]
        r = jnp.maximum(t, 0.0)
        o_ref[...] = dinv_ref[...] * jnp.dot(r, w_ref[...],
                                             preferred_element_type=jnp.float32)

    n = g1.shape[0]
    return pl.pallas_call(
        body,
        grid=(n // _BR,),
        in_specs=[
            pl.BlockSpec((_BR, D), lambda i: (i, 0)),
            pl.BlockSpec((_BR, D), lambda i: (i, 0)),
            pl.BlockSpec((_BR, D), lambda i: (i, 0)),
            pl.BlockSpec((_BR, 1), lambda i: (i, 0)),
            pl.BlockSpec((1, D), lambda i: (0, 0)),
            pl.BlockSpec((D, D), lambda i: (0, 0)),
        ],
        out_specs=pl.BlockSpec((_BR, D), lambda i: (i, 0)),
        out_shape=jax.ShapeDtypeStruct((n, D), jnp.float32),
    )(p0, p1, g1, dinv, b1, w2)


def _tc_final(q0, q1, g2, dinv, b2):
    def body(q0_ref, q1_ref, g_ref, dinv_ref, b_ref, o_ref):
        o_ref[...] = dinv_ref[...] * (
            q0_ref[...] + q1_ref[...] + g_ref[...]) + b_ref[...]

    n = g2.shape[0]
    return pl.pallas_call(
        body,
        grid=(n // _BR,),
        in_specs=[
            pl.BlockSpec((_BR, D), lambda i: (i, 0)),
            pl.BlockSpec((_BR, D), lambda i: (i, 0)),
            pl.BlockSpec((_BR, D), lambda i: (i, 0)),
            pl.BlockSpec((_BR, 1), lambda i: (i, 0)),
            pl.BlockSpec((1, D), lambda i: (0, 0)),
        ],
        out_specs=pl.BlockSpec((_BR, D), lambda i: (i, 0)),
        out_shape=jax.ShapeDtypeStruct((n, D), jnp.float32),
    )(q0, q1, g2, dinv, b2)


def kernel(x, edge_index, W1, b1, W2, b2):
    ei = edge_index.astype(jnp.int32)
    j0, j1 = _edge_layout(ei.shape[1])
    srcp, dstp = _pad_edges(ei[0], ei[1], j0, j1)

    counts = _sc_degree(dstp)
    c0 = counts[0, :N].reshape(N, 1)
    c1 = counts[1, :N].reshape(N, 1)

    g1, dinv = _tc_scale_matmul(x, c0, c1, W1)
    p = _sc_scatter(g1, srcp, dstp, j0, j1)
    g2 = _tc_mid(p[0, :N], p[1, :N], g1, dinv, b1.reshape(1, D), W2)
    q = _sc_scatter(g2, srcp, dstp, j0, j1)
    return _tc_final(q[0, :N], q[1, :N], g2, dinv, b2.reshape(1, D))


# speedup vs baseline: 2.0357x; 1.0295x over previous
import functools

import jax
import jax.numpy as jnp
from jax import lax
from jax.experimental import pallas as pl
from jax.experimental.pallas import tpu as pltpu
from jax.experimental.pallas import tpu_sc as plsc

N = 10000
D = 128
NC = 2
NS = 16
NW = NC * NS
CH = 64
F0 = 0.83
CHD = 128

ACC_ROWS = 10112
SEG = ACC_ROWS // NS
COUNT_PAD = 10240
CSEG = COUNT_PAD // NS

_NB = 4


def _edge_layout(e):
    chunks = -(-e // CH)
    j0 = max(-(-int(chunks * F0) // NS), 2 * _NB)
    j0 = -(-j0 // _NB) * _NB
    j1 = max(-(-max(chunks - NS * j0, 0) // NS), 2 * _NB)
    j1 = -(-j1 // _NB) * _NB
    return j0, j1


def _pad_edges(src, dst, j0, j1):
    e = src.shape[0]
    totc = NS * (j0 + j1)
    pad = totc * CH - e
    srcp = jnp.concatenate([src, jnp.zeros((pad,), jnp.int32)]).reshape(totc, CH)
    dstp = jnp.concatenate([dst, jnp.full((pad,), N, jnp.int32)]).reshape(totc, CH)
    return srcp, dstp


def _mesh():
    return plsc.VectorSubcoreMesh(core_axis_name="c", subcore_axis_name="s")


def _sc_degree(dst_flat):
    njd = dst_flat.shape[0] // NW
    assert dst_flat.shape[0] % NW == 0 and njd >= 4

    @functools.partial(
        pl.kernel,
        mesh=_mesh(),
        out_type=jax.ShapeDtypeStruct((NC, COUNT_PAD), jnp.float32),
        scratch_types=(
            [pltpu.VMEM((CHD,), jnp.float32)]
            + [pltpu.VMEM((CHD,), jnp.int32) for _ in range(3)]
            + [pltpu.SemaphoreType.DMA for _ in range(6)]
            + [pltpu.VMEM((CSEG,), jnp.float32),
               pltpu.VMEM_SHARED((COUNT_PAD,), jnp.float32)]
        ),
    )
    def deg_k(dst_hbm, out_hbm, ones_v, *refs):
        idx = refs[0:3]
        isem = refs[3:6]
        ssem = refs[6:9]
        zrow_v, counts = refs[9], refs[10]
        cid = lax.axis_index("c")
        sid = lax.axis_index("s")
        base = (sid * NC + cid) * njd
        z16 = jnp.zeros((16,), jnp.float32)
        o16 = jnp.ones((16,), jnp.float32)
        for k in range(CHD // 16):
            ones_v[pl.ds(k * 16, 16)] = o16

        def zb(i, c):
            zrow_v[pl.ds(i * 16, 16)] = z16
            return c

        lax.fori_loop(0, CSEG // 16, zb, 0)
        pltpu.sync_copy(zrow_v, counts.at[pl.ds(sid * CSEG, CSEG)])
        plsc.subcore_barrier()

        def idx_start(b, j):
            pltpu.async_copy(dst_hbm.at[base + j], idx[b], isem[b])

        def visit(b, j, warm, prefetch):
            pltpu.make_async_copy(dst_hbm.at[base + j], idx[b], isem[b]).wait()
            if warm:
                pltpu.make_async_copy(ones_v, counts.at[idx[(b + 1) % 3]],
                                      ssem[(b + 1) % 3]).wait()
            if prefetch:
                idx_start((b + 1) % 3, j + 1)
            pltpu.async_copy(ones_v, counts.at[idx[b]], ssem[b], add=True)

        idx_start(0, 0)
        visit(0, 0, False, True)
        visit(1, 1, False, True)

        nq = (njd - 3) // 3

        def steady(i, c):
            for r in range(3):
                visit((2 + r) % 3, 2 + i * 3 + r, True, True)
            return c

        lax.fori_loop(0, nq, steady, 0)
        for v in range(2 + 3 * nq, njd):
            visit(v % 3, v, True, v + 1 < njd)
        for s in ((njd - 2) % 3, (njd - 1) % 3):
            pltpu.make_async_copy(ones_v, counts.at[idx[s]], ssem[s]).wait()

        plsc.subcore_barrier()
        pltpu.sync_copy(counts.at[pl.ds(sid * CSEG, CSEG)],
                        out_hbm.at[cid, pl.ds(sid * CSEG, CSEG)])

    return deg_k(dst_flat)


def _sc_scatter(table, srcp, dstp, j0, j1):
    for nj in (j0, j1):
        assert nj % _NB == 0 and nj >= 2 * _NB

    @functools.partial(
        pl.kernel,
        mesh=_mesh(),
        out_type=jax.ShapeDtypeStruct((NC, ACC_ROWS, D), jnp.float32),
        scratch_types=(
            [pltpu.VMEM((CH,), jnp.int32) for _ in range(2 * _NB)]
            + [pltpu.VMEM((CH, D), jnp.float32) for _ in range(_NB)]
            + [pltpu.SemaphoreType.DMA for _ in range(3 * _NB)]
            + [pltpu.VMEM_SHARED((ACC_ROWS, D), jnp.float32)]
        ),
    )
    def scat_k(tab_hbm, src_hbm, dst_hbm, out_hbm, *refs):
        sidx = refs[0:_NB]
        didx = refs[_NB:2 * _NB]
        rows = refs[2 * _NB:3 * _NB]
        isem = refs[3 * _NB:4 * _NB]
        gsem = refs[4 * _NB:5 * _NB]
        ssem = refs[5 * _NB:6 * _NB]
        acc = refs[6 * _NB]
        cid = lax.axis_index("c")
        sid = lax.axis_index("s")
        z16 = jnp.zeros((16,), jnp.float32)

        def zrow(r, c):
            for k in range(D // 16):
                rows[0][r, pl.ds(k * 16, 16)] = z16
            return c

        lax.fori_loop(0, CH, zrow, 0)
        base_r = sid * SEG
        nfull, rem = SEG // CH, SEG % CH
        for c in range(nfull):
            pltpu.sync_copy(rows[0], acc.at[pl.ds(base_r + c * CH, CH)])
        if rem:
            pltpu.sync_copy(rows[0].at[pl.ds(0, rem)],
                            acc.at[pl.ds(base_r + nfull * CH, rem)])
        plsc.subcore_barrier()

        def pipe(nj, base):
            def idx_start(s, j):
                pltpu.async_copy(src_hbm.at[base + j], sidx[s], isem[s])
                pltpu.async_copy(dst_hbm.at[base + j], didx[s], isem[s])

            def idx_wait(s, j):
                pltpu.make_async_copy(src_hbm.at[base + j], sidx[s],
                                      isem[s]).wait()
                pltpu.make_async_copy(dst_hbm.at[base + j], didx[s],
                                      isem[s]).wait()

            def gat_start(s):
                pltpu.async_copy(tab_hbm.at[sidx[s]], rows[s], gsem[s])

            def gat_wait(s):
                pltpu.make_async_copy(tab_hbm.at[sidx[s]], rows[s],
                                      gsem[s]).wait()

            def sca_start(s):
                pltpu.async_copy(rows[s], acc.at[didx[s]], ssem[s], add=True)

            def sca_wait(s):
                pltpu.make_async_copy(rows[s], acc.at[didx[s]],
                                      ssem[s]).wait()

            idx_start(0, 0)
            for v in range(3):
                idx_wait(v, v)
                gat_start(v)
                idx_start(v + 1, v + 1)
                if v >= 2:
                    gat_wait((v + 2) % _NB)
                    sca_start((v + 2) % _NB)

            def steady(i, c):
                v0 = 3 + i * _NB
                for r in range(_NB):
                    b = (3 + r) % _NB
                    v = v0 + r
                    idx_wait(b, v)
                    gat_start(b)
                    sca_wait((b + 1) % _NB)
                    idx_start((b + 1) % _NB, v + 1)
                    gat_wait((b + 2) % _NB)
                    sca_start((b + 2) % _NB)
                return c

            lax.fori_loop(0, (nj - 2 - 3 + 1) // _NB, steady, 0)

            bl = (nj - 1) % _NB
            idx_wait(bl, nj - 1)
            gat_start(bl)
            for v in (nj - 3, nj - 2, nj - 1):
                s = v % _NB
                gat_wait(s)
                sca_start(s)
            for s in range(_NB):
                sca_wait(s)

        @pl.when(cid == 0)
        def _():
            pipe(j0, sid * j0)

        @pl.when(cid == 1)
        def _():
            pipe(j1, NS * j0 + sid * j1)

        plsc.subcore_barrier()
        pltpu.sync_copy(acc.at[pl.ds(base_r, SEG)],
                        out_hbm.at[cid, pl.ds(base_r, SEG)])

    return scat_k(table, srcp, dstp)


_BR = 1000


def _tc_scale_matmul(x, c0, c1, w1):
    def body(x_ref, c0_ref, c1_ref, w_ref, g_ref, dinv_ref):
        dinv = lax.rsqrt(c0_ref[...] + c1_ref[...] + 1.0)
        g_ref[...] = jnp.dot(x_ref[...] * dinv, w_ref[...],
                             preferred_element_type=jnp.float32)
        dinv_ref[...] = dinv

    n = x.shape[0]
    return pl.pallas_call(
        body,
        grid=(n // _BR,),
        in_specs=[
            pl.BlockSpec((_BR, D), lambda i: (i, 0)),
            pl.BlockSpec((_BR, 1), lambda i: (i, 0)),
            pl.BlockSpec((_BR, 1), lambda i: (i, 0)),
            pl.BlockSpec((D, D), lambda i: (0, 0)),
        ],
        out_specs=[
            pl.BlockSpec((_BR, D), lambda i: (i, 0)),
            pl.BlockSpec((_BR, 1), lambda i: (i, 0)),
        ],
        out_shape=[
            jax.ShapeDtypeStruct((n, D), jnp.float32),
            jax.ShapeDtypeStruct((n, 1), jnp.float32),
        ],
    )(x, c0, c1, w1)


def _tc_mid(p0, p1, g1, dinv, b1, w2):
    def body(p0_ref, p1_ref, g_ref, dinv_ref, b_ref, w_ref, o_ref):
        t = dinv_ref[...] * (p0_ref[...] + p1_ref[...] + g_ref[...]) + b_ref[...]
        r = jnp.maximum(t, 0.0)
        o_ref[...] = dinv_ref[...] * jnp.dot(r, w_ref[...],
                                             preferred_element_type=jnp.float32)

    n = g1.shape[0]
    return pl.pallas_call(
        body,
        grid=(n // _BR,),
        in_specs=[
            pl.BlockSpec((_BR, D), lambda i: (i, 0)),
            pl.BlockSpec((_BR, D), lambda i: (i, 0)),
            pl.BlockSpec((_BR, D), lambda i: (i, 0)),
            pl.BlockSpec((_BR, 1), lambda i: (i, 0)),
            pl.BlockSpec((1, D), lambda i: (0, 0)),
            pl.BlockSpec((D, D), lambda i: (0, 0)),
        ],
        out_specs=pl.BlockSpec((_BR, D), lambda i: (i, 0)),
        out_shape=jax.ShapeDtypeStruct((n, D), jnp.float32),
    )(p0, p1, g1, dinv, b1, w2)


def _tc_final(q0, q1, g2, dinv, b2):
    def body(q0_ref, q1_ref, g_ref, dinv_ref, b_ref, o_ref):
        o_ref[...] = dinv_ref[...] * (
            q0_ref[...] + q1_ref[...] + g_ref[...]) + b_ref[...]

    n = g2.shape[0]
    return pl.pallas_call(
        body,
        grid=(n // _BR,),
        in_specs=[
            pl.BlockSpec((_BR, D), lambda i: (i, 0)),
            pl.BlockSpec((_BR, D), lambda i: (i, 0)),
            pl.BlockSpec((_BR, D), lambda i: (i, 0)),
            pl.BlockSpec((_BR, 1), lambda i: (i, 0)),
            pl.BlockSpec((1, D), lambda i: (0, 0)),
        ],
        out_specs=pl.BlockSpec((_BR, D), lambda i: (i, 0)),
        out_shape=jax.ShapeDtypeStruct((n, D), jnp.float32),
    )(q0, q1, g2, dinv, b2)


def kernel(x, edge_index, W1, b1, W2, b2):
    ei = edge_index.astype(jnp.int32)
    j0, j1 = _edge_layout(ei.shape[1])
    srcp, dstp = _pad_edges(ei[0], ei[1], j0, j1)

    counts = _sc_degree(dstp.reshape(-1, CHD))
    c0 = counts[0, :N].reshape(N, 1)
    c1 = counts[1, :N].reshape(N, 1)

    g1, dinv = _tc_scale_matmul(x, c0, c1, W1)
    p = _sc_scatter(g1, srcp, dstp, j0, j1)
    g2 = _tc_mid(p[0, :N], p[1, :N], g1, dinv, b1.reshape(1, D), W2)
    q = _sc_scatter(g2, srcp, dstp, j0, j1)
    return _tc_final(q[0, :N], q[1, :N], g2, dinv, b2.reshape(1, D))


# speedup vs baseline: 2.6410x; 1.2974x over previous
import functools

import jax
import jax.numpy as jnp
from jax import lax
from jax.experimental import pallas as pl
from jax.experimental.pallas import tpu as pltpu
from jax.experimental.pallas import tpu_sc as plsc

N = 10000
D = 128
NC = 2
NS = 16
NW = NC * NS
CH = 64
CHD = 128
F0 = 0.88

COUNT_PAD = 10240
CSEG = COUNT_PAD // NS

_BLK8 = N // 8
_LEN_LO = (_BLK8 // NS) * 8
_EXTRA = _BLK8 % NS
_SPLIT = NS - _EXTRA

_NB = 4


def _mesh():
    return plsc.VectorSubcoreMesh(core_axis_name="c", subcore_axis_name="s")


def _sc_degree(dst_flat, njd, njd_last):
    @functools.partial(
        pl.kernel,
        mesh=_mesh(),
        out_type=jax.ShapeDtypeStruct((NC, COUNT_PAD), jnp.float32),
        scratch_types=(
            [pltpu.VMEM((CHD,), jnp.float32)]
            + [pltpu.VMEM((CHD,), jnp.int32) for _ in range(3)]
            + [pltpu.SemaphoreType.DMA for _ in range(6)]
            + [pltpu.VMEM((CSEG,), jnp.float32),
               pltpu.VMEM_SHARED((COUNT_PAD,), jnp.float32)]
        ),
    )
    def deg_k(dst_hbm, out_hbm, ones_v, *refs):
        idx = refs[0:3]
        isem = refs[3:6]
        ssem = refs[6:9]
        zrow_v, counts = refs[9], refs[10]
        cid = lax.axis_index("c")
        sid = lax.axis_index("s")
        wid = sid * NC + cid
        z16 = jnp.zeros((16,), jnp.float32)
        o16 = jnp.ones((16,), jnp.float32)
        for k in range(CHD // 16):
            ones_v[pl.ds(k * 16, 16)] = o16

        def zb(i, c):
            zrow_v[pl.ds(i * 16, 16)] = z16
            return c

        lax.fori_loop(0, CSEG // 16, zb, 0)
        pltpu.sync_copy(zrow_v, counts.at[pl.ds(sid * CSEG, CSEG)])
        plsc.subcore_barrier()

        def dpipe(nj, base):
            def idx_start(b, j):
                pltpu.async_copy(dst_hbm.at[base + j], idx[b], isem[b])

            def visit(b, j, warm, prefetch):
                pltpu.make_async_copy(dst_hbm.at[base + j], idx[b],
                                      isem[b]).wait()
                if warm:
                    pltpu.make_async_copy(ones_v, counts.at[idx[(b + 1) % 3]],
                                          ssem[(b + 1) % 3]).wait()
                if prefetch:
                    idx_start((b + 1) % 3, j + 1)
                pltpu.async_copy(ones_v, counts.at[idx[b]], ssem[b], add=True)

            idx_start(0, 0)
            visit(0, 0, False, True)
            visit(1, 1, False, True)

            nq = (nj - 3) // 3

            def steady(i, c):
                for r in range(3):
                    visit((2 + r) % 3, 2 + i * 3 + r, True, True)
                return c

            lax.fori_loop(0, nq, steady, 0)
            for v in range(2 + 3 * nq, nj):
                visit(v % 3, v, True, v + 1 < nj)
            for s in ((nj - 2) % 3, (nj - 1) % 3):
                pltpu.make_async_copy(ones_v, counts.at[idx[s]],
                                      ssem[s]).wait()

        @pl.when(wid < NW - 1)
        def _():
            dpipe(njd, wid * njd)

        @pl.when(wid == NW - 1)
        def _():
            dpipe(njd_last, (NW - 1) * njd)

        plsc.subcore_barrier()
        pltpu.sync_copy(counts.at[pl.ds(sid * CSEG, CSEG)],
                        out_hbm.at[cid, pl.ds(sid * CSEG, CSEG)])

    return deg_k(dst_flat)


def _sc_scatter(table, srcp, dstp, j0, j1, j1t):
    for nj in (j0, j1, j1t):
        assert nj >= _NB

    @functools.partial(
        pl.kernel,
        mesh=_mesh(),
        out_type=jax.ShapeDtypeStruct((NC, N, D), jnp.float32),
        scratch_types=(
            [pltpu.VMEM((CH,), jnp.int32) for _ in range(2 * _NB)]
            + [pltpu.VMEM((CH, D), jnp.float32) for _ in range(_NB)]
            + [pltpu.SemaphoreType.DMA for _ in range(3 * _NB)]
            + [pltpu.VMEM_SHARED((N, D), jnp.float32)]
        ),
    )
    def scat_k(tab_hbm, src_hbm, dst_hbm, out_hbm, *refs):
        sidx = refs[0:_NB]
        didx = refs[_NB:2 * _NB]
        rows = refs[2 * _NB:3 * _NB]
        isem = refs[3 * _NB:4 * _NB]
        gsem = refs[4 * _NB:5 * _NB]
        ssem = refs[5 * _NB:6 * _NB]
        acc = refs[6 * _NB]
        cid = lax.axis_index("c")
        sid = lax.axis_index("s")
        z16 = jnp.zeros((16,), jnp.float32)

        def zrow(r, c):
            for k in range(D // 16):
                rows[0][r, pl.ds(k * 16, 16)] = z16
            return c

        lax.fori_loop(0, CH, zrow, 0)
        base_r = sid * _LEN_LO + 8 * jnp.maximum(sid - _SPLIT, 0)

        def zfill(nrows):
            nfull, rem = nrows // CH, nrows % CH
            for c in range(nfull):
                pltpu.sync_copy(rows[0], acc.at[pl.ds(base_r + c * CH, CH)])
            if rem:
                pltpu.sync_copy(rows[0].at[pl.ds(0, rem)],
                                acc.at[pl.ds(base_r + nfull * CH, rem)])

        @pl.when(sid < _SPLIT)
        def _():
            zfill(_LEN_LO)

        @pl.when(sid >= _SPLIT)
        def _():
            zfill(_LEN_LO + 8)

        plsc.subcore_barrier()

        def pipe(nj, base):
            def idx_start(s, j):
                pltpu.async_copy(src_hbm.at[base + j], sidx[s], isem[s])
                pltpu.async_copy(dst_hbm.at[base + j], didx[s], isem[s])

            def idx_wait(s, j):
                pltpu.make_async_copy(src_hbm.at[base + j], sidx[s],
                                      isem[s]).wait()
                pltpu.make_async_copy(dst_hbm.at[base + j], didx[s],
                                      isem[s]).wait()

            def gat_start(s):
                pltpu.async_copy(tab_hbm.at[sidx[s]], rows[s], gsem[s])

            def gat_wait(s):
                pltpu.make_async_copy(tab_hbm.at[sidx[s]], rows[s],
                                      gsem[s]).wait()

            def sca_start(s):
                pltpu.async_copy(rows[s], acc.at[didx[s]], ssem[s], add=True)

            def sca_wait(s):
                pltpu.make_async_copy(rows[s], acc.at[didx[s]],
                                      ssem[s]).wait()

            idx_start(0, 0)
            for v in range(3):
                idx_wait(v, v)
                gat_start(v)
                idx_start((v + 1) % _NB, min(v + 1, nj - 1))
                if v >= 2:
                    gat_wait((v + 2) % _NB)
                    sca_start((v + 2) % _NB)

            nq = (nj - 4) // _NB

            def steady(i, c):
                v0 = 3 + i * _NB
                for r in range(_NB):
                    b = (3 + r) % _NB
                    v = v0 + r
                    idx_wait(b, v)
                    gat_start(b)
                    sca_wait((b + 1) % _NB)
                    idx_start((b + 1) % _NB, v + 1)
                    gat_wait((b + 2) % _NB)
                    sca_start((b + 2) % _NB)
                return c

            lax.fori_loop(0, nq, steady, 0)

            for v in range(3 + _NB * nq, nj - 1):
                b = v % _NB
                idx_wait(b, v)
                gat_start(b)
                sca_wait((b + 1) % _NB)
                idx_start((b + 1) % _NB, v + 1)
                gat_wait((b + 2) % _NB)
                sca_start((b + 2) % _NB)

            bl = (nj - 1) % _NB
            idx_wait(bl, nj - 1)
            gat_start(bl)
            sca_wait((bl + 1) % _NB)
            for v in (nj - 3, nj - 2, nj - 1):
                s = v % _NB
                gat_wait(s)
                sca_start(s)
                sca_wait(s)

        @pl.when(cid == 0)
        def _():
            pipe(j0, sid * j0)

        @pl.when(jnp.logical_and(cid == 1, sid < NS - 1))
        def _():
            pipe(j1, NS * j0 + sid * j1)

        @pl.when(jnp.logical_and(cid == 1, sid == NS - 1))
        def _():
            pipe(j1t, NS * j0 + (NS - 1) * j1)

        plsc.subcore_barrier()

        @pl.when(sid < _SPLIT)
        def _():
            pltpu.sync_copy(acc.at[pl.ds(base_r, _LEN_LO)],
                            out_hbm.at[cid, pl.ds(base_r, _LEN_LO)])

        @pl.when(sid >= _SPLIT)
        def _():
            pltpu.sync_copy(acc.at[pl.ds(base_r, _LEN_LO + 8)],
                            out_hbm.at[cid, pl.ds(base_r, _LEN_LO + 8)])

    return scat_k(table, srcp, dstp)


_BR = 1000


def _tc_scale_matmul(x, c0, c1, base, w1):
    def body(x_ref, c0_ref, c1_ref, base_ref, w_ref, g_ref, dinv_ref):
        dinv = lax.rsqrt(c0_ref[...] + c1_ref[...] + base_ref[...])
        g_ref[...] = jnp.dot(x_ref[...] * dinv, w_ref[...],
                             preferred_element_type=jnp.float32)
        dinv_ref[...] = dinv

    n = x.shape[0]
    return pl.pallas_call(
        body,
        grid=(n // _BR,),
        in_specs=[
            pl.BlockSpec((_BR, D), lambda i: (i, 0)),
            pl.BlockSpec((_BR, 1), lambda i: (i, 0)),
            pl.BlockSpec((_BR, 1), lambda i: (i, 0)),
            pl.BlockSpec((_BR, 1), lambda i: (i, 0)),
            pl.BlockSpec((D, D), lambda i: (0, 0)),
        ],
        out_specs=[
            pl.BlockSpec((_BR, D), lambda i: (i, 0)),
            pl.BlockSpec((_BR, 1), lambda i: (i, 0)),
        ],
        out_shape=[
            jax.ShapeDtypeStruct((n, D), jnp.float32),
            jax.ShapeDtypeStruct((n, 1), jnp.float32),
        ],
    )(x, c0, c1, base, w1)


def _tc_mid(p, g1, dinv, w, b1, w2):
    def body(p0_ref, p1_ref, g_ref, dinv_ref, w_ref, b_ref, w2_ref, o_ref):
        t = dinv_ref[...] * (p0_ref[0] + p1_ref[0]
                             + w_ref[...] * g_ref[...]) + b_ref[...]
        r = jnp.maximum(t, 0.0)
        o_ref[...] = dinv_ref[...] * jnp.dot(r, w2_ref[...],
                                             preferred_element_type=jnp.float32)

    n = g1.shape[0]
    return pl.pallas_call(
        body,
        grid=(n // _BR,),
        in_specs=[
            pl.BlockSpec((1, _BR, D), lambda i: (0, i, 0)),
            pl.BlockSpec((1, _BR, D), lambda i: (1, i, 0)),
            pl.BlockSpec((_BR, D), lambda i: (i, 0)),
            pl.BlockSpec((_BR, 1), lambda i: (i, 0)),
            pl.BlockSpec((_BR, 1), lambda i: (i, 0)),
            pl.BlockSpec((1, D), lambda i: (0, 0)),
            pl.BlockSpec((D, D), lambda i: (0, 0)),
        ],
        out_specs=pl.BlockSpec((_BR, D), lambda i: (i, 0)),
        out_shape=jax.ShapeDtypeStruct((n, D), jnp.float32),
    )(p, p, g1, dinv, w, b1, w2)


def _tc_final(q, g2, dinv, w, b2):
    def body(q0_ref, q1_ref, g_ref, dinv_ref, w_ref, b_ref, o_ref):
        o_ref[...] = dinv_ref[...] * (q0_ref[0] + q1_ref[0]
                                      + w_ref[...] * g_ref[...]) + b_ref[...]

    n = g2.shape[0]
    return pl.pallas_call(
        body,
        grid=(n // _BR,),
        in_specs=[
            pl.BlockSpec((1, _BR, D), lambda i: (0, i, 0)),
            pl.BlockSpec((1, _BR, D), lambda i: (1, i, 0)),
            pl.BlockSpec((_BR, D), lambda i: (i, 0)),
            pl.BlockSpec((_BR, 1), lambda i: (i, 0)),
            pl.BlockSpec((_BR, 1), lambda i: (i, 0)),
            pl.BlockSpec((1, D), lambda i: (0, 0)),
        ],
        out_specs=pl.BlockSpec((_BR, D), lambda i: (i, 0)),
        out_shape=jax.ShapeDtypeStruct((n, D), jnp.float32),
    )(q, q, g2, dinv, w, b2)


def kernel(x, edge_index, W1, b1, W2, b2):
    ei = edge_index.astype(jnp.int32)
    e = ei.shape[1]

    npad = (-e) % CHD
    if npad:
        ei = jnp.concatenate(
            [ei, jnp.zeros((2, npad), jnp.int32)], axis=1)
    ep = e + npad

    chunks = ep // CH
    j0 = max(_NB, int(chunks * F0) // NS)
    j0 = min(j0, (chunks - NS * _NB) // NS)
    rem = chunks - NS * j0
    j1 = max(_NB, rem // NS)
    j1t = rem - (NS - 1) * j1
    assert j1t >= _NB

    dchunks = ep // CHD
    njd = dchunks // NW
    njd_last = dchunks - (NW - 1) * njd
    assert njd >= 4

    srcp = ei[0].reshape(chunks, CH)
    dstp = ei[1].reshape(chunks, CH)

    counts = _sc_degree(ei[1].reshape(dchunks, CHD), njd, njd_last)
    c0 = counts[0, :N].reshape(N, 1)
    c1 = counts[1, :N].reshape(N, 1)

    base = jnp.ones((N, 1), jnp.float32).at[0, 0].add(-float(npad))
    w = jnp.ones((N, 1), jnp.float32).at[0, 0].add(-float(npad))

    g1, dinv = _tc_scale_matmul(x, c0, c1, base, W1)
    p = _sc_scatter(g1, srcp, dstp, j0, j1, j1t)
    g2 = _tc_mid(p, g1, dinv, w, b1.reshape(1, D), W2)
    q = _sc_scatter(g2, srcp, dstp, j0, j1, j1t)
    return _tc_final(q, g2, dinv, w, b2.reshape(1, D))


# speedup vs baseline: 3.0251x; 1.1454x over previous
import functools

import jax
import jax.numpy as jnp
from jax import lax
from jax.experimental import pallas as pl
from jax.experimental.pallas import tpu as pltpu
from jax.experimental.pallas import tpu_sc as plsc

N = 10000
D = 128
NC = 2
NS = 16
NW = NC * NS
CH = 64
CHD = 128
F0 = 0.80

COUNT_PAD = 10240
CSEG = COUNT_PAD // NS

_BLK8 = N // 8
_LEN_LO = (_BLK8 // NS) * 8
_EXTRA = _BLK8 % NS
_SPLIT = NS - _EXTRA

_NB = 4


def _mesh():
    return plsc.VectorSubcoreMesh(core_axis_name="c", subcore_axis_name="s")


def _sc_degree(eflat, ep, njd, njd_last):
    @functools.partial(
        pl.kernel,
        mesh=_mesh(),
        out_type=jax.ShapeDtypeStruct((NC, COUNT_PAD), jnp.float32),
        scratch_types=(
            [pltpu.VMEM((CHD,), jnp.float32)]
            + [pltpu.VMEM((CHD,), jnp.int32) for _ in range(3)]
            + [pltpu.SemaphoreType.DMA for _ in range(6)]
            + [pltpu.VMEM((CSEG,), jnp.float32),
               pltpu.VMEM_SHARED((COUNT_PAD,), jnp.float32)]
        ),
    )
    def deg_k(dst_hbm, out_hbm, ones_v, *refs):
        idx = refs[0:3]
        isem = refs[3:6]
        ssem = refs[6:9]
        zrow_v, counts = refs[9], refs[10]
        cid = lax.axis_index("c")
        sid = lax.axis_index("s")
        wid = sid * NC + cid
        z16 = jnp.zeros((16,), jnp.float32)
        o16 = jnp.ones((16,), jnp.float32)
        for k in range(CHD // 16):
            ones_v[pl.ds(k * 16, 16)] = o16

        def zb(i, c):
            zrow_v[pl.ds(i * 16, 16)] = z16
            return c

        lax.fori_loop(0, CSEG // 16, zb, 0)
        pltpu.sync_copy(zrow_v, counts.at[pl.ds(sid * CSEG, CSEG)])
        plsc.subcore_barrier()

        def dpipe(nj, base):
            def dslc(j):
                return dst_hbm.at[pl.ds(ep + (base + j) * CHD, CHD)]

            def idx_start(b, j):
                pltpu.async_copy(dslc(j), idx[b], isem[b])

            def visit(b, j, warm, prefetch):
                pltpu.make_async_copy(dslc(j), idx[b], isem[b]).wait()
                if warm:
                    pltpu.make_async_copy(ones_v, counts.at[idx[(b + 1) % 3]],
                                          ssem[(b + 1) % 3]).wait()
                if prefetch:
                    idx_start((b + 1) % 3, j + 1)
                pltpu.async_copy(ones_v, counts.at[idx[b]], ssem[b], add=True)

            idx_start(0, 0)
            visit(0, 0, False, True)
            visit(1, 1, False, True)

            nq = (nj - 3) // 3

            def steady(i, c):
                for r in range(3):
                    visit((2 + r) % 3, 2 + i * 3 + r, True, True)
                return c

            lax.fori_loop(0, nq, steady, 0)
            for v in range(2 + 3 * nq, nj):
                visit(v % 3, v, True, v + 1 < nj)
            for s in ((nj - 2) % 3, (nj - 1) % 3):
                pltpu.make_async_copy(ones_v, counts.at[idx[s]],
                                      ssem[s]).wait()

        @pl.when(wid < NW - 1)
        def _():
            dpipe(njd, wid * njd)

        @pl.when(wid == NW - 1)
        def _():
            dpipe(njd_last, (NW - 1) * njd)

        plsc.subcore_barrier()
        pltpu.sync_copy(counts.at[pl.ds(sid * CSEG, CSEG)],
                        out_hbm.at[cid, pl.ds(sid * CSEG, CSEG)])

    return deg_k(eflat)


def _sc_scatter(table, eflat, ep, j0, j1, j1t):
    for nj in (j0, j1, j1t):
        assert nj >= _NB

    @functools.partial(
        pl.kernel,
        mesh=_mesh(),
        out_type=jax.ShapeDtypeStruct((NC, N, D), jnp.float32),
        scratch_types=(
            [pltpu.VMEM((CH,), jnp.int32) for _ in range(2 * _NB)]
            + [pltpu.VMEM((CH, D), jnp.float32) for _ in range(_NB)]
            + [pltpu.SemaphoreType.DMA for _ in range(3 * _NB)]
            + [pltpu.VMEM_SHARED((N, D), jnp.float32)]
        ),
    )
    def scat_k(tab_hbm, e_hbm, out_hbm, *refs):
        sidx = refs[0:_NB]
        didx = refs[_NB:2 * _NB]
        rows = refs[2 * _NB:3 * _NB]
        isem = refs[3 * _NB:4 * _NB]
        gsem = refs[4 * _NB:5 * _NB]
        ssem = refs[5 * _NB:6 * _NB]
        acc = refs[6 * _NB]
        cid = lax.axis_index("c")
        sid = lax.axis_index("s")
        z16 = jnp.zeros((16,), jnp.float32)

        def zrow(r, c):
            for k in range(D // 16):
                rows[0][r, pl.ds(k * 16, 16)] = z16
            return c

        lax.fori_loop(0, CH, zrow, 0)
        base_r = sid * _LEN_LO + 8 * jnp.maximum(sid - _SPLIT, 0)

        def zfill(nrows):
            nfull, rem = nrows // CH, nrows % CH
            for c in range(nfull):
                pltpu.sync_copy(rows[0], acc.at[pl.ds(base_r + c * CH, CH)])
            if rem:
                pltpu.sync_copy(rows[0].at[pl.ds(0, rem)],
                                acc.at[pl.ds(base_r + nfull * CH, rem)])

        @pl.when(sid < _SPLIT)
        def _():
            zfill(_LEN_LO)

        @pl.when(sid >= _SPLIT)
        def _():
            zfill(_LEN_LO + 8)

        plsc.subcore_barrier()

        def pipe(nj, base):
            def sslc(j):
                return e_hbm.at[pl.ds((base + j) * CH, CH)]

            def dslc(j):
                return e_hbm.at[pl.ds(ep + (base + j) * CH, CH)]

            def idx_start(s, j):
                pltpu.async_copy(sslc(j), sidx[s], isem[s])
                pltpu.async_copy(dslc(j), didx[s], isem[s])

            def idx_wait(s, j):
                pltpu.make_async_copy(sslc(j), sidx[s], isem[s]).wait()
                pltpu.make_async_copy(dslc(j), didx[s], isem[s]).wait()

            def gat_start(s):
                pltpu.async_copy(tab_hbm.at[sidx[s]], rows[s], gsem[s])

            def gat_wait(s):
                pltpu.make_async_copy(tab_hbm.at[sidx[s]], rows[s],
                                      gsem[s]).wait()

            def sca_start(s):
                pltpu.async_copy(rows[s], acc.at[didx[s]], ssem[s], add=True)

            def sca_wait(s):
                pltpu.make_async_copy(rows[s], acc.at[didx[s]],
                                      ssem[s]).wait()

            idx_start(0, 0)
            for v in range(3):
                idx_wait(v, v)
                gat_start(v)
                idx_start((v + 1) % _NB, min(v + 1, nj - 1))
                if v >= 2:
                    gat_wait((v + 2) % _NB)
                    sca_start((v + 2) % _NB)

            nq = (nj - 4) // _NB

            def steady(i, c):
                v0 = 3 + i * _NB
                for r in range(_NB):
                    b = (3 + r) % _NB
                    v = v0 + r
                    idx_wait(b, v)
                    gat_start(b)
                    sca_wait((b + 1) % _NB)
                    idx_start((b + 1) % _NB, v + 1)
                    gat_wait((b + 2) % _NB)
                    sca_start((b + 2) % _NB)
                return c

            lax.fori_loop(0, nq, steady, 0)

            for v in range(3 + _NB * nq, nj - 1):
                b = v % _NB
                idx_wait(b, v)
                gat_start(b)
                sca_wait((b + 1) % _NB)
                idx_start((b + 1) % _NB, v + 1)
                gat_wait((b + 2) % _NB)
                sca_start((b + 2) % _NB)

            bl = (nj - 1) % _NB
            idx_wait(bl, nj - 1)
            gat_start(bl)
            sca_wait((bl + 1) % _NB)
            for v in (nj - 3, nj - 2, nj - 1):
                s = v % _NB
                gat_wait(s)
                sca_start(s)
                sca_wait(s)

        @pl.when(cid == 0)
        def _():
            pipe(j0, sid * j0)

        @pl.when(jnp.logical_and(cid == 1, sid < NS - 1))
        def _():
            pipe(j1, NS * j0 + sid * j1)

        @pl.when(jnp.logical_and(cid == 1, sid == NS - 1))
        def _():
            pipe(j1t, NS * j0 + (NS - 1) * j1)

        plsc.subcore_barrier()

        @pl.when(sid < _SPLIT)
        def _():
            pltpu.sync_copy(acc.at[pl.ds(base_r, _LEN_LO)],
                            out_hbm.at[cid, pl.ds(base_r, _LEN_LO)])

        @pl.when(sid >= _SPLIT)
        def _():
            pltpu.sync_copy(acc.at[pl.ds(base_r, _LEN_LO + 8)],
                            out_hbm.at[cid, pl.ds(base_r, _LEN_LO + 8)])

    return scat_k(table, eflat)


_BR = 1000


def _tc_scale_matmul(x, c0, c1, base, w1):
    def body(x_ref, c0_ref, c1_ref, base_ref, w_ref, g_ref, dinv_ref):
        dinv = lax.rsqrt(c0_ref[...] + c1_ref[...] + base_ref[...])
        g_ref[...] = jnp.dot(x_ref[...] * dinv, w_ref[...],
                             preferred_element_type=jnp.float32)
        dinv_ref[...] = dinv

    n = x.shape[0]
    return pl.pallas_call(
        body,
        grid=(n // _BR,),
        in_specs=[
            pl.BlockSpec((_BR, D), lambda i: (i, 0)),
            pl.BlockSpec((_BR, 1), lambda i: (i, 0)),
            pl.BlockSpec((_BR, 1), lambda i: (i, 0)),
            pl.BlockSpec((_BR, 1), lambda i: (i, 0)),
            pl.BlockSpec((D, D), lambda i: (0, 0)),
        ],
        out_specs=[
            pl.BlockSpec((_BR, D), lambda i: (i, 0)),
            pl.BlockSpec((_BR, 1), lambda i: (i, 0)),
        ],
        out_shape=[
            jax.ShapeDtypeStruct((n, D), jnp.float32),
            jax.ShapeDtypeStruct((n, 1), jnp.float32),
        ],
    )(x, c0, c1, base, w1)


def _tc_mid(p, g1, dinv, w, b1, w2):
    def body(p0_ref, p1_ref, g_ref, dinv_ref, w_ref, b_ref, w2_ref, o_ref):
        t = dinv_ref[...] * (p0_ref[0] + p1_ref[0]
                             + w_ref[...] * g_ref[...]) + b_ref[...]
        r = jnp.maximum(t, 0.0)
        o_ref[...] = dinv_ref[...] * jnp.dot(r, w2_ref[...],
                                             preferred_element_type=jnp.float32)

    n = g1.shape[0]
    return pl.pallas_call(
        body,
        grid=(n // _BR,),
        in_specs=[
            pl.BlockSpec((1, _BR, D), lambda i: (0, i, 0)),
            pl.BlockSpec((1, _BR, D), lambda i: (1, i, 0)),
            pl.BlockSpec((_BR, D), lambda i: (i, 0)),
            pl.BlockSpec((_BR, 1), lambda i: (i, 0)),
            pl.BlockSpec((_BR, 1), lambda i: (i, 0)),
            pl.BlockSpec((1, D), lambda i: (0, 0)),
            pl.BlockSpec((D, D), lambda i: (0, 0)),
        ],
        out_specs=pl.BlockSpec((_BR, D), lambda i: (i, 0)),
        out_shape=jax.ShapeDtypeStruct((n, D), jnp.float32),
    )(p, p, g1, dinv, w, b1, w2)


def _tc_final(q, g2, dinv, w, b2):
    def body(q0_ref, q1_ref, g_ref, dinv_ref, w_ref, b_ref, o_ref):
        o_ref[...] = dinv_ref[...] * (q0_ref[0] + q1_ref[0]
                                      + w_ref[...] * g_ref[...]) + b_ref[...]

    n = g2.shape[0]
    return pl.pallas_call(
        body,
        grid=(n // _BR,),
        in_specs=[
            pl.BlockSpec((1, _BR, D), lambda i: (0, i, 0)),
            pl.BlockSpec((1, _BR, D), lambda i: (1, i, 0)),
            pl.BlockSpec((_BR, D), lambda i: (i, 0)),
            pl.BlockSpec((_BR, 1), lambda i: (i, 0)),
            pl.BlockSpec((_BR, 1), lambda i: (i, 0)),
            pl.BlockSpec((1, D), lambda i: (0, 0)),
        ],
        out_specs=pl.BlockSpec((_BR, D), lambda i: (i, 0)),
        out_shape=jax.ShapeDtypeStruct((n, D), jnp.float32),
    )(q, q, g2, dinv, w, b2)


def kernel(x, edge_index, W1, b1, W2, b2):
    ei = edge_index.astype(jnp.int32)
    e = ei.shape[1]

    npad = (-e) % CHD
    if npad:
        ei = jnp.concatenate(
            [ei, jnp.zeros((2, npad), jnp.int32)], axis=1)
    ep = e + npad

    chunks = ep // CH
    j0 = max(_NB, int(chunks * F0) // NS)
    j0 = min(j0, (chunks - NS * _NB) // NS)
    rem = chunks - NS * j0
    j1 = max(_NB, rem // NS)
    j1t = rem - (NS - 1) * j1
    assert j1t >= _NB

    dchunks = ep // CHD
    njd = dchunks // NW
    njd_last = dchunks - (NW - 1) * njd
    assert njd >= 4

    eflat = ei.reshape(2 * ep)

    counts = _sc_degree(eflat, ep, njd, njd_last)
    c0 = counts[0, :N].reshape(N, 1)
    c1 = counts[1, :N].reshape(N, 1)

    base = jnp.ones((N, 1), jnp.float32).at[0, 0].add(-float(npad))
    w = jnp.ones((N, 1), jnp.float32).at[0, 0].add(-float(npad))

    g1, dinv = _tc_scale_matmul(x, c0, c1, base, W1)
    p = _sc_scatter(g1, eflat, ep, j0, j1, j1t)
    g2 = _tc_mid(p, g1, dinv, w, b1.reshape(1, D), W2)
    q = _sc_scatter(g2, eflat, ep, j0, j1, j1t)
    return _tc_final(q, g2, dinv, w, b2.reshape(1, D))


# speedup vs baseline: 3.2221x; 1.0651x over previous
import functools

import jax
import jax.numpy as jnp
from jax import lax
from jax.experimental import pallas as pl
from jax.experimental.pallas import tpu as pltpu
from jax.experimental.pallas import tpu_sc as plsc

N = 10000
D = 128
NC = 2
NS = 16
NW = NC * NS
CH = 64
CHD = 128
F0 = 0.72

COUNT_PAD = 10240
CSEG = COUNT_PAD // NS

_BLK8 = N // 8
_LEN_LO = (_BLK8 // NS) * 8
_EXTRA = _BLK8 % NS
_SPLIT = NS - _EXTRA

_NB = 4


def _mesh():
    return plsc.VectorSubcoreMesh(core_axis_name="c", subcore_axis_name="s")


def _sc_degree(eflat, ep, njd, njd_last):
    @functools.partial(
        pl.kernel,
        mesh=_mesh(),
        out_type=jax.ShapeDtypeStruct((NC, COUNT_PAD), jnp.float32),
        scratch_types=(
            [pltpu.VMEM((CHD,), jnp.float32)]
            + [pltpu.VMEM((CHD,), jnp.int32) for _ in range(3)]
            + [pltpu.SemaphoreType.DMA for _ in range(6)]
            + [pltpu.VMEM((CSEG,), jnp.float32),
               pltpu.VMEM_SHARED((COUNT_PAD,), jnp.float32)]
        ),
    )
    def deg_k(dst_hbm, out_hbm, ones_v, *refs):
        idx = refs[0:3]
        isem = refs[3:6]
        ssem = refs[6:9]
        zrow_v, counts = refs[9], refs[10]
        cid = lax.axis_index("c")
        sid = lax.axis_index("s")
        wid = sid * NC + cid
        z16 = jnp.zeros((16,), jnp.float32)
        o16 = jnp.ones((16,), jnp.float32)
        for k in range(CHD // 16):
            ones_v[pl.ds(k * 16, 16)] = o16

        def zb(i, c):
            zrow_v[pl.ds(i * 16, 16)] = z16
            return c

        lax.fori_loop(0, CSEG // 16, zb, 0)
        pltpu.sync_copy(zrow_v, counts.at[pl.ds(sid * CSEG, CSEG)])
        plsc.subcore_barrier()

        def dpipe(nj, base):
            def dslc(j):
                return dst_hbm.at[pl.ds(ep + (base + j) * CHD, CHD)]

            def idx_start(b, j):
                pltpu.async_copy(dslc(j), idx[b], isem[b])

            def visit(b, j, warm, prefetch):
                pltpu.make_async_copy(dslc(j), idx[b], isem[b]).wait()
                if warm:
                    pltpu.make_async_copy(ones_v, counts.at[idx[(b + 1) % 3]],
                                          ssem[(b + 1) % 3]).wait()
                if prefetch:
                    idx_start((b + 1) % 3, j + 1)
                pltpu.async_copy(ones_v, counts.at[idx[b]], ssem[b], add=True)

            idx_start(0, 0)
            visit(0, 0, False, True)
            visit(1, 1, False, True)

            nq = (nj - 3) // 3

            def steady(i, c):
                for r in range(3):
                    visit((2 + r) % 3, 2 + i * 3 + r, True, True)
                return c

            lax.fori_loop(0, nq, steady, 0)
            for v in range(2 + 3 * nq, nj):
                visit(v % 3, v, True, v + 1 < nj)
            for s in ((nj - 2) % 3, (nj - 1) % 3):
                pltpu.make_async_copy(ones_v, counts.at[idx[s]],
                                      ssem[s]).wait()

        @pl.when(wid < NW - 1)
        def _():
            dpipe(njd, wid * njd)

        @pl.when(wid == NW - 1)
        def _():
            dpipe(njd_last, (NW - 1) * njd)

        plsc.subcore_barrier()
        pltpu.sync_copy(counts.at[pl.ds(sid * CSEG, CSEG)],
                        out_hbm.at[cid, pl.ds(sid * CSEG, CSEG)])

    return deg_k(eflat)


def _sc_scatter(table, eflat, ep, j0, j1, j1t):
    for nj in (j0, j1, j1t):
        assert nj >= _NB

    @functools.partial(
        pl.kernel,
        mesh=_mesh(),
        out_type=jax.ShapeDtypeStruct((NC, N, D), jnp.float32),
        scratch_types=(
            [pltpu.VMEM((CH,), jnp.int32) for _ in range(2 * _NB)]
            + [pltpu.VMEM((CH, D), jnp.float32) for _ in range(_NB)]
            + [pltpu.SemaphoreType.DMA for _ in range(3 * _NB)]
            + [pltpu.VMEM_SHARED((N, D), jnp.float32)]
        ),
    )
    def scat_k(tab_hbm, e_hbm, out_hbm, *refs):
        sidx = refs[0:_NB]
        didx = refs[_NB:2 * _NB]
        rows = refs[2 * _NB:3 * _NB]
        isem = refs[3 * _NB:4 * _NB]
        gsem = refs[4 * _NB:5 * _NB]
        ssem = refs[5 * _NB:6 * _NB]
        acc = refs[6 * _NB]
        cid = lax.axis_index("c")
        sid = lax.axis_index("s")
        z16 = jnp.zeros((16,), jnp.float32)

        def zrow(r, c):
            for k in range(D // 16):
                rows[0][r, pl.ds(k * 16, 16)] = z16
            return c

        lax.fori_loop(0, CH, zrow, 0)
        base_r = sid * _LEN_LO + 8 * jnp.maximum(sid - _SPLIT, 0)

        def zfill(nrows):
            nfull, rem = nrows // CH, nrows % CH
            for c in range(nfull):
                pltpu.sync_copy(rows[0], acc.at[pl.ds(base_r + c * CH, CH)])
            if rem:
                pltpu.sync_copy(rows[0].at[pl.ds(0, rem)],
                                acc.at[pl.ds(base_r + nfull * CH, rem)])

        @pl.when(sid < _SPLIT)
        def _():
            zfill(_LEN_LO)

        @pl.when(sid >= _SPLIT)
        def _():
            zfill(_LEN_LO + 8)

        plsc.subcore_barrier()

        def pipe(nj, base):
            def sslc(j):
                return e_hbm.at[pl.ds((base + j) * CH, CH)]

            def dslc(j):
                return e_hbm.at[pl.ds(ep + (base + j) * CH, CH)]

            def idx_start(s, j):
                pltpu.async_copy(sslc(j), sidx[s], isem[s])
                pltpu.async_copy(dslc(j), didx[s], isem[s])

            def idx_wait(s, j):
                pltpu.make_async_copy(sslc(j), sidx[s], isem[s]).wait()
                pltpu.make_async_copy(dslc(j), didx[s], isem[s]).wait()

            def gat_start(s):
                pltpu.async_copy(tab_hbm.at[sidx[s]], rows[s], gsem[s])

            def gat_wait(s):
                pltpu.make_async_copy(tab_hbm.at[sidx[s]], rows[s],
                                      gsem[s]).wait()

            def sca_start(s):
                pltpu.async_copy(rows[s], acc.at[didx[s]], ssem[s], add=True)

            def sca_wait(s):
                pltpu.make_async_copy(rows[s], acc.at[didx[s]],
                                      ssem[s]).wait()

            idx_start(0, 0)
            for v in range(3):
                idx_wait(v, v)
                gat_start(v)
                idx_start((v + 1) % _NB, min(v + 1, nj - 1))
                if v >= 2:
                    gat_wait((v + 2) % _NB)
                    sca_start((v + 2) % _NB)

            nq = (nj - 4) // _NB

            def steady(i, c):
                v0 = 3 + i * _NB
                for r in range(_NB):
                    b = (3 + r) % _NB
                    v = v0 + r
                    idx_wait(b, v)
                    gat_start(b)
                    sca_wait((b + 1) % _NB)
                    idx_start((b + 1) % _NB, v + 1)
                    gat_wait((b + 2) % _NB)
                    sca_start((b + 2) % _NB)
                return c

            lax.fori_loop(0, nq, steady, 0)

            for v in range(3 + _NB * nq, nj - 1):
                b = v % _NB
                idx_wait(b, v)
                gat_start(b)
                sca_wait((b + 1) % _NB)
                idx_start((b + 1) % _NB, v + 1)
                gat_wait((b + 2) % _NB)
                sca_start((b + 2) % _NB)

            bl = (nj - 1) % _NB
            idx_wait(bl, nj - 1)
            gat_start(bl)
            sca_wait((bl + 1) % _NB)
            for v in (nj - 3, nj - 2, nj - 1):
                s = v % _NB
                gat_wait(s)
                sca_start(s)
                sca_wait(s)

        @pl.when(cid == 0)
        def _():
            pipe(j0, sid * j0)

        @pl.when(jnp.logical_and(cid == 1, sid < NS - 1))
        def _():
            pipe(j1, NS * j0 + sid * j1)

        @pl.when(jnp.logical_and(cid == 1, sid == NS - 1))
        def _():
            pipe(j1t, NS * j0 + (NS - 1) * j1)

        plsc.subcore_barrier()

        @pl.when(sid < _SPLIT)
        def _():
            pltpu.sync_copy(acc.at[pl.ds(base_r, _LEN_LO)],
                            out_hbm.at[cid, pl.ds(base_r, _LEN_LO)])

        @pl.when(sid >= _SPLIT)
        def _():
            pltpu.sync_copy(acc.at[pl.ds(base_r, _LEN_LO + 8)],
                            out_hbm.at[cid, pl.ds(base_r, _LEN_LO + 8)])

    return scat_k(table, eflat)


_BR = 1000


def _tc_scale_matmul(x, c0, c1, base, w1):
    def body(x_ref, c0_ref, c1_ref, base_ref, w_ref, g_ref, dinv_ref):
        dinv = lax.rsqrt(c0_ref[...] + c1_ref[...] + base_ref[...])
        g_ref[...] = jnp.dot(x_ref[...] * dinv, w_ref[...],
                             preferred_element_type=jnp.float32)
        dinv_ref[...] = dinv

    n = x.shape[0]
    return pl.pallas_call(
        body,
        grid=(n // _BR,),
        in_specs=[
            pl.BlockSpec((_BR, D), lambda i: (i, 0)),
            pl.BlockSpec((_BR, 1), lambda i: (i, 0)),
            pl.BlockSpec((_BR, 1), lambda i: (i, 0)),
            pl.BlockSpec((_BR, 1), lambda i: (i, 0)),
            pl.BlockSpec((D, D), lambda i: (0, 0)),
        ],
        out_specs=[
            pl.BlockSpec((_BR, D), lambda i: (i, 0)),
            pl.BlockSpec((_BR, 1), lambda i: (i, 0)),
        ],
        out_shape=[
            jax.ShapeDtypeStruct((n, D), jnp.float32),
            jax.ShapeDtypeStruct((n, 1), jnp.float32),
        ],
    )(x, c0, c1, base, w1)


def _tc_mid(p, g1, dinv, w, b1, w2):
    def body(p0_ref, p1_ref, g_ref, dinv_ref, w_ref, b_ref, w2_ref, o_ref):
        t = dinv_ref[...] * (p0_ref[0] + p1_ref[0]
                             + w_ref[...] * g_ref[...]) + b_ref[...]
        r = jnp.maximum(t, 0.0)
        o_ref[...] = dinv_ref[...] * jnp.dot(r, w2_ref[...],
                                             preferred_element_type=jnp.float32)

    n = g1.shape[0]
    return pl.pallas_call(
        body,
        grid=(n // _BR,),
        in_specs=[
            pl.BlockSpec((1, _BR, D), lambda i: (0, i, 0)),
            pl.BlockSpec((1, _BR, D), lambda i: (1, i, 0)),
            pl.BlockSpec((_BR, D), lambda i: (i, 0)),
            pl.BlockSpec((_BR, 1), lambda i: (i, 0)),
            pl.BlockSpec((_BR, 1), lambda i: (i, 0)),
            pl.BlockSpec((1, D), lambda i: (0, 0)),
            pl.BlockSpec((D, D), lambda i: (0, 0)),
        ],
        out_specs=pl.BlockSpec((_BR, D), lambda i: (i, 0)),
        out_shape=jax.ShapeDtypeStruct((n, D), jnp.float32),
    )(p, p, g1, dinv, w, b1, w2)


def _tc_final(q, g2, dinv, w, b2):
    def body(q0_ref, q1_ref, g_ref, dinv_ref, w_ref, b_ref, o_ref):
        o_ref[...] = dinv_ref[...] * (q0_ref[0] + q1_ref[0]
                                      + w_ref[...] * g_ref[...]) + b_ref[...]

    n = g2.shape[0]
    return pl.pallas_call(
        body,
        grid=(n // _BR,),
        in_specs=[
            pl.BlockSpec((1, _BR, D), lambda i: (0, i, 0)),
            pl.BlockSpec((1, _BR, D), lambda i: (1, i, 0)),
            pl.BlockSpec((_BR, D), lambda i: (i, 0)),
            pl.BlockSpec((_BR, 1), lambda i: (i, 0)),
            pl.BlockSpec((_BR, 1), lambda i: (i, 0)),
            pl.BlockSpec((1, D), lambda i: (0, 0)),
        ],
        out_specs=pl.BlockSpec((_BR, D), lambda i: (i, 0)),
        out_shape=jax.ShapeDtypeStruct((n, D), jnp.float32),
    )(q, q, g2, dinv, w, b2)


def kernel(x, edge_index, W1, b1, W2, b2):
    ei = edge_index.astype(jnp.int32)
    e = ei.shape[1]

    npad = (-e) % CHD
    if npad:
        ei = jnp.concatenate(
            [ei, jnp.zeros((2, npad), jnp.int32)], axis=1)
    ep = e + npad

    chunks = ep // CH
    j0 = max(_NB, int(chunks * F0) // NS)
    j0 = min(j0, (chunks - NS * _NB) // NS)
    rem = chunks - NS * j0
    j1 = max(_NB, rem // NS)
    j1t = rem - (NS - 1) * j1
    assert j1t >= _NB

    dchunks = ep // CHD
    njd = dchunks // NW
    njd_last = dchunks - (NW - 1) * njd
    assert njd >= 4

    eflat = ei.reshape(2 * ep)

    counts = _sc_degree(eflat, ep, njd, njd_last)
    c0 = counts[0, :N].reshape(N, 1)
    c1 = counts[1, :N].reshape(N, 1)

    base = jnp.ones((N, 1), jnp.float32).at[0, 0].add(-float(npad))
    w = jnp.ones((N, 1), jnp.float32).at[0, 0].add(-float(npad))

    g1, dinv = _tc_scale_matmul(x, c0, c1, base, W1)
    p = _sc_scatter(g1, eflat, ep, j0, j1, j1t)
    g2 = _tc_mid(p, g1, dinv, w, b1.reshape(1, D), W2)
    q = _sc_scatter(g2, eflat, ep, j0, j1, j1t)
    return _tc_final(q, g2, dinv, w, b2.reshape(1, D))


# speedup vs baseline: 3.4562x; 1.0727x over previous
import functools

import jax
import jax.numpy as jnp
from jax import lax
from jax.experimental import pallas as pl
from jax.experimental.pallas import tpu as pltpu
from jax.experimental.pallas import tpu_sc as plsc

N = 10000
D = 128
NC = 2
NS = 16
NW = NC * NS
CH = 64
CHD = 128
F0 = 0.64

COUNT_PAD = 10240
CSEG = COUNT_PAD // NS

_BLK8 = N // 8
_LEN_LO = (_BLK8 // NS) * 8
_EXTRA = _BLK8 % NS
_SPLIT = NS - _EXTRA

_NB = 4


def _mesh():
    return plsc.VectorSubcoreMesh(core_axis_name="c", subcore_axis_name="s")


def _sc_degree(eflat, ep, njd, njd_last):
    @functools.partial(
        pl.kernel,
        mesh=_mesh(),
        out_type=jax.ShapeDtypeStruct((NC, COUNT_PAD), jnp.float32),
        scratch_types=(
            [pltpu.VMEM((CHD,), jnp.float32)]
            + [pltpu.VMEM((CHD,), jnp.int32) for _ in range(3)]
            + [pltpu.SemaphoreType.DMA for _ in range(6)]
            + [pltpu.VMEM((CSEG,), jnp.float32),
               pltpu.VMEM_SHARED((COUNT_PAD,), jnp.float32)]
        ),
    )
    def deg_k(dst_hbm, out_hbm, ones_v, *refs):
        idx = refs[0:3]
        isem = refs[3:6]
        ssem = refs[6:9]
        zrow_v, counts = refs[9], refs[10]
        cid = lax.axis_index("c")
        sid = lax.axis_index("s")
        wid = sid * NC + cid
        z16 = jnp.zeros((16,), jnp.float32)
        o16 = jnp.ones((16,), jnp.float32)
        for k in range(CHD // 16):
            ones_v[pl.ds(k * 16, 16)] = o16

        def zb(i, c):
            zrow_v[pl.ds(i * 16, 16)] = z16
            return c

        lax.fori_loop(0, CSEG // 16, zb, 0)
        pltpu.sync_copy(zrow_v, counts.at[pl.ds(sid * CSEG, CSEG)])
        plsc.subcore_barrier()

        def dpipe(nj, base):
            def dslc(j):
                return dst_hbm.at[pl.ds(ep + (base + j) * CHD, CHD)]

            def idx_start(b, j):
                pltpu.async_copy(dslc(j), idx[b], isem[b])

            def visit(b, j, warm, prefetch):
                pltpu.make_async_copy(dslc(j), idx[b], isem[b]).wait()
                if warm:
                    pltpu.make_async_copy(ones_v, counts.at[idx[(b + 1) % 3]],
                                          ssem[(b + 1) % 3]).wait()
                if prefetch:
                    idx_start((b + 1) % 3, j + 1)
                pltpu.async_copy(ones_v, counts.at[idx[b]], ssem[b], add=True)

            idx_start(0, 0)
            visit(0, 0, False, True)
            visit(1, 1, False, True)

            nq = (nj - 3) // 3

            def steady(i, c):
                for r in range(3):
                    visit((2 + r) % 3, 2 + i * 3 + r, True, True)
                return c

            lax.fori_loop(0, nq, steady, 0)
            for v in range(2 + 3 * nq, nj):
                visit(v % 3, v, True, v + 1 < nj)
            for s in ((nj - 2) % 3, (nj - 1) % 3):
                pltpu.make_async_copy(ones_v, counts.at[idx[s]],
                                      ssem[s]).wait()

        @pl.when(wid < NW - 1)
        def _():
            dpipe(njd, wid * njd)

        @pl.when(wid == NW - 1)
        def _():
            dpipe(njd_last, (NW - 1) * njd)

        plsc.subcore_barrier()
        pltpu.sync_copy(counts.at[pl.ds(sid * CSEG, CSEG)],
                        out_hbm.at[cid, pl.ds(sid * CSEG, CSEG)])

    return deg_k(eflat)


def _sc_scatter(table, eflat, ep, j0, j1, j1t):
    for nj in (j0, j1, j1t):
        assert nj >= _NB

    @functools.partial(
        pl.kernel,
        mesh=_mesh(),
        out_type=jax.ShapeDtypeStruct((NC, N, D), jnp.float32),
        scratch_types=(
            [pltpu.VMEM((CH,), jnp.int32) for _ in range(2 * _NB)]
            + [pltpu.VMEM((CH, D), jnp.float32) for _ in range(_NB)]
            + [pltpu.SemaphoreType.DMA for _ in range(3 * _NB)]
            + [pltpu.VMEM_SHARED((N, D), jnp.float32)]
        ),
    )
    def scat_k(tab_hbm, e_hbm, out_hbm, *refs):
        sidx = refs[0:_NB]
        didx = refs[_NB:2 * _NB]
        rows = refs[2 * _NB:3 * _NB]
        isem = refs[3 * _NB:4 * _NB]
        gsem = refs[4 * _NB:5 * _NB]
        ssem = refs[5 * _NB:6 * _NB]
        acc = refs[6 * _NB]
        cid = lax.axis_index("c")
        sid = lax.axis_index("s")
        z16 = jnp.zeros((16,), jnp.float32)

        def zrow(r, c):
            for k in range(D // 16):
                rows[0][r, pl.ds(k * 16, 16)] = z16
            return c

        lax.fori_loop(0, CH, zrow, 0)
        base_r = sid * _LEN_LO + 8 * jnp.maximum(sid - _SPLIT, 0)

        def zfill(nrows):
            nfull, rem = nrows // CH, nrows % CH
            for c in range(nfull):
                pltpu.sync_copy(rows[0], acc.at[pl.ds(base_r + c * CH, CH)])
            if rem:
                pltpu.sync_copy(rows[0].at[pl.ds(0, rem)],
                                acc.at[pl.ds(base_r + nfull * CH, rem)])

        @pl.when(sid < _SPLIT)
        def _():
            zfill(_LEN_LO)

        @pl.when(sid >= _SPLIT)
        def _():
            zfill(_LEN_LO + 8)

        plsc.subcore_barrier()

        def pipe(nj, base):
            def sslc(j):
                return e_hbm.at[pl.ds((base + j) * CH, CH)]

            def dslc(j):
                return e_hbm.at[pl.ds(ep + (base + j) * CH, CH)]

            def idx_start(s, j):
                pltpu.async_copy(sslc(j), sidx[s], isem[s])
                pltpu.async_copy(dslc(j), didx[s], isem[s])

            def idx_wait(s, j):
                pltpu.make_async_copy(sslc(j), sidx[s], isem[s]).wait()
                pltpu.make_async_copy(dslc(j), didx[s], isem[s]).wait()

            def gat_start(s):
                pltpu.async_copy(tab_hbm.at[sidx[s]], rows[s], gsem[s])

            def gat_wait(s):
                pltpu.make_async_copy(tab_hbm.at[sidx[s]], rows[s],
                                      gsem[s]).wait()

            def sca_start(s):
                pltpu.async_copy(rows[s], acc.at[didx[s]], ssem[s], add=True)

            def sca_wait(s):
                pltpu.make_async_copy(rows[s], acc.at[didx[s]],
                                      ssem[s]).wait()

            idx_start(0, 0)
            for v in range(3):
                idx_wait(v, v)
                gat_start(v)
                idx_start((v + 1) % _NB, min(v + 1, nj - 1))
                if v >= 2:
                    gat_wait((v + 2) % _NB)
                    sca_start((v + 2) % _NB)

            nq = (nj - 4) // _NB

            def steady(i, c):
                v0 = 3 + i * _NB
                for r in range(_NB):
                    b = (3 + r) % _NB
                    v = v0 + r
                    idx_wait(b, v)
                    gat_start(b)
                    sca_wait((b + 1) % _NB)
                    idx_start((b + 1) % _NB, v + 1)
                    gat_wait((b + 2) % _NB)
                    sca_start((b + 2) % _NB)
                return c

            lax.fori_loop(0, nq, steady, 0)

            for v in range(3 + _NB * nq, nj - 1):
                b = v % _NB
                idx_wait(b, v)
                gat_start(b)
                sca_wait((b + 1) % _NB)
                idx_start((b + 1) % _NB, v + 1)
                gat_wait((b + 2) % _NB)
                sca_start((b + 2) % _NB)

            bl = (nj - 1) % _NB
            idx_wait(bl, nj - 1)
            gat_start(bl)
            sca_wait((bl + 1) % _NB)
            for v in (nj - 3, nj - 2, nj - 1):
                s = v % _NB
                gat_wait(s)
                sca_start(s)
                sca_wait(s)

        @pl.when(cid == 0)
        def _():
            pipe(j0, sid * j0)

        @pl.when(jnp.logical_and(cid == 1, sid < NS - 1))
        def _():
            pipe(j1, NS * j0 + sid * j1)

        @pl.when(jnp.logical_and(cid == 1, sid == NS - 1))
        def _():
            pipe(j1t, NS * j0 + (NS - 1) * j1)

        plsc.subcore_barrier()

        @pl.when(sid < _SPLIT)
        def _():
            pltpu.sync_copy(acc.at[pl.ds(base_r, _LEN_LO)],
                            out_hbm.at[cid, pl.ds(base_r, _LEN_LO)])

        @pl.when(sid >= _SPLIT)
        def _():
            pltpu.sync_copy(acc.at[pl.ds(base_r, _LEN_LO + 8)],
                            out_hbm.at[cid, pl.ds(base_r, _LEN_LO + 8)])

    return scat_k(table, eflat)


_BR = 1000


def _tc_scale_matmul(x, c0, c1, base, w1):
    def body(x_ref, c0_ref, c1_ref, base_ref, w_ref, g_ref, dinv_ref):
        dinv = lax.rsqrt(c0_ref[...] + c1_ref[...] + base_ref[...])
        g_ref[...] = jnp.dot(x_ref[...] * dinv, w_ref[...],
                             preferred_element_type=jnp.float32)
        dinv_ref[...] = dinv

    n = x.shape[0]
    return pl.pallas_call(
        body,
        grid=(n // _BR,),
        in_specs=[
            pl.BlockSpec((_BR, D), lambda i: (i, 0)),
            pl.BlockSpec((_BR, 1), lambda i: (i, 0)),
            pl.BlockSpec((_BR, 1), lambda i: (i, 0)),
            pl.BlockSpec((_BR, 1), lambda i: (i, 0)),
            pl.BlockSpec((D, D), lambda i: (0, 0)),
        ],
        out_specs=[
            pl.BlockSpec((_BR, D), lambda i: (i, 0)),
            pl.BlockSpec((_BR, 1), lambda i: (i, 0)),
        ],
        out_shape=[
            jax.ShapeDtypeStruct((n, D), jnp.float32),
            jax.ShapeDtypeStruct((n, 1), jnp.float32),
        ],
    )(x, c0, c1, base, w1)


def _tc_mid(p, g1, dinv, w, b1, w2):
    def body(p0_ref, p1_ref, g_ref, dinv_ref, w_ref, b_ref, w2_ref, o_ref):
        t = dinv_ref[...] * (p0_ref[0] + p1_ref[0]
                             + w_ref[...] * g_ref[...]) + b_ref[...]
        r = jnp.maximum(t, 0.0)
        o_ref[...] = dinv_ref[...] * jnp.dot(r, w2_ref[...],
                                             preferred_element_type=jnp.float32)

    n = g1.shape[0]
    return pl.pallas_call(
        body,
        grid=(n // _BR,),
        in_specs=[
            pl.BlockSpec((1, _BR, D), lambda i: (0, i, 0)),
            pl.BlockSpec((1, _BR, D), lambda i: (1, i, 0)),
            pl.BlockSpec((_BR, D), lambda i: (i, 0)),
            pl.BlockSpec((_BR, 1), lambda i: (i, 0)),
            pl.BlockSpec((_BR, 1), lambda i: (i, 0)),
            pl.BlockSpec((1, D), lambda i: (0, 0)),
            pl.BlockSpec((D, D), lambda i: (0, 0)),
        ],
        out_specs=pl.BlockSpec((_BR, D), lambda i: (i, 0)),
        out_shape=jax.ShapeDtypeStruct((n, D), jnp.float32),
    )(p, p, g1, dinv, w, b1, w2)


def _tc_final(q, g2, dinv, w, b2):
    def body(q0_ref, q1_ref, g_ref, dinv_ref, w_ref, b_ref, o_ref):
        o_ref[...] = dinv_ref[...] * (q0_ref[0] + q1_ref[0]
                                      + w_ref[...] * g_ref[...]) + b_ref[...]

    n = g2.shape[0]
    return pl.pallas_call(
        body,
        grid=(n // _BR,),
        in_specs=[
            pl.BlockSpec((1, _BR, D), lambda i: (0, i, 0)),
            pl.BlockSpec((1, _BR, D), lambda i: (1, i, 0)),
            pl.BlockSpec((_BR, D), lambda i: (i, 0)),
            pl.BlockSpec((_BR, 1), lambda i: (i, 0)),
            pl.BlockSpec((_BR, 1), lambda i: (i, 0)),
            pl.BlockSpec((1, D), lambda i: (0, 0)),
        ],
        out_specs=pl.BlockSpec((_BR, D), lambda i: (i, 0)),
        out_shape=jax.ShapeDtypeStruct((n, D), jnp.float32),
    )(q, q, g2, dinv, w, b2)


def kernel(x, edge_index, W1, b1, W2, b2):
    ei = edge_index.astype(jnp.int32)
    e = ei.shape[1]

    npad = (-e) % CHD
    if npad:
        ei = jnp.concatenate(
            [ei, jnp.zeros((2, npad), jnp.int32)], axis=1)
    ep = e + npad

    chunks = ep // CH
    j0 = max(_NB, int(chunks * F0) // NS)
    j0 = min(j0, (chunks - NS * _NB) // NS)
    rem = chunks - NS * j0
    j1 = max(_NB, rem // NS)
    j1t = rem - (NS - 1) * j1
    assert j1t >= _NB

    dchunks = ep // CHD
    njd = dchunks // NW
    njd_last = dchunks - (NW - 1) * njd
    assert njd >= 4

    eflat = ei.reshape(2 * ep)

    counts = _sc_degree(eflat, ep, njd, njd_last)
    c0 = counts[0, :N].reshape(N, 1)
    c1 = counts[1, :N].reshape(N, 1)

    base = jnp.ones((N, 1), jnp.float32).at[0, 0].add(-float(npad))
    w = jnp.ones((N, 1), jnp.float32).at[0, 0].add(-float(npad))

    g1, dinv = _tc_scale_matmul(x, c0, c1, base, W1)
    p = _sc_scatter(g1, eflat, ep, j0, j1, j1t)
    g2 = _tc_mid(p, g1, dinv, w, b1.reshape(1, D), W2)
    q = _sc_scatter(g2, eflat, ep, j0, j1, j1t)
    return _tc_final(q, g2, dinv, w, b2.reshape(1, D))


# speedup vs baseline: 3.7107x; 1.0736x over previous
import functools

import jax
import jax.numpy as jnp
from jax import lax
from jax.experimental import pallas as pl
from jax.experimental.pallas import tpu as pltpu
from jax.experimental.pallas import tpu_sc as plsc

N = 10000
D = 128
NC = 2
NS = 16
NW = NC * NS
CH = 64
CHD = 128
F0 = 0.56

COUNT_PAD = 10240
CSEG = COUNT_PAD // NS

_BLK8 = N // 8
_LEN_LO = (_BLK8 // NS) * 8
_EXTRA = _BLK8 % NS
_SPLIT = NS - _EXTRA

_NB = 4


def _mesh():
    return plsc.VectorSubcoreMesh(core_axis_name="c", subcore_axis_name="s")


def _sc_degree(eflat, ep, njd, njd_last):
    @functools.partial(
        pl.kernel,
        mesh=_mesh(),
        out_type=jax.ShapeDtypeStruct((NC, COUNT_PAD), jnp.float32),
        scratch_types=(
            [pltpu.VMEM((CHD,), jnp.float32)]
            + [pltpu.VMEM((CHD,), jnp.int32) for _ in range(3)]
            + [pltpu.SemaphoreType.DMA for _ in range(6)]
            + [pltpu.VMEM((CSEG,), jnp.float32),
               pltpu.VMEM_SHARED((COUNT_PAD,), jnp.float32)]
        ),
    )
    def deg_k(dst_hbm, out_hbm, ones_v, *refs):
        idx = refs[0:3]
        isem = refs[3:6]
        ssem = refs[6:9]
        zrow_v, counts = refs[9], refs[10]
        cid = lax.axis_index("c")
        sid = lax.axis_index("s")
        wid = sid * NC + cid
        z16 = jnp.zeros((16,), jnp.float32)
        o16 = jnp.ones((16,), jnp.float32)
        for k in range(CHD // 16):
            ones_v[pl.ds(k * 16, 16)] = o16

        def zb(i, c):
            zrow_v[pl.ds(i * 16, 16)] = z16
            return c

        lax.fori_loop(0, CSEG // 16, zb, 0)
        pltpu.sync_copy(zrow_v, counts.at[pl.ds(sid * CSEG, CSEG)])
        plsc.subcore_barrier()

        def dpipe(nj, base):
            def dslc(j):
                return dst_hbm.at[pl.ds(ep + (base + j) * CHD, CHD)]

            def idx_start(b, j):
                pltpu.async_copy(dslc(j), idx[b], isem[b])

            def visit(b, j, warm, prefetch):
                pltpu.make_async_copy(dslc(j), idx[b], isem[b]).wait()
                if warm:
                    pltpu.make_async_copy(ones_v, counts.at[idx[(b + 1) % 3]],
                                          ssem[(b + 1) % 3]).wait()
                if prefetch:
                    idx_start((b + 1) % 3, j + 1)
                pltpu.async_copy(ones_v, counts.at[idx[b]], ssem[b], add=True)

            idx_start(0, 0)
            visit(0, 0, False, True)
            visit(1, 1, False, True)

            nq = (nj - 3) // 3

            def steady(i, c):
                for r in range(3):
                    visit((2 + r) % 3, 2 + i * 3 + r, True, True)
                return c

            lax.fori_loop(0, nq, steady, 0)
            for v in range(2 + 3 * nq, nj):
                visit(v % 3, v, True, v + 1 < nj)
            for s in ((nj - 2) % 3, (nj - 1) % 3):
                pltpu.make_async_copy(ones_v, counts.at[idx[s]],
                                      ssem[s]).wait()

        @pl.when(wid < NW - 1)
        def _():
            dpipe(njd, wid * njd)

        @pl.when(wid == NW - 1)
        def _():
            dpipe(njd_last, (NW - 1) * njd)

        plsc.subcore_barrier()
        pltpu.sync_copy(counts.at[pl.ds(sid * CSEG, CSEG)],
                        out_hbm.at[cid, pl.ds(sid * CSEG, CSEG)])

    return deg_k(eflat)


def _sc_scatter(table, eflat, ep, j0, j1, j1t):
    for nj in (j0, j1, j1t):
        assert nj >= _NB

    @functools.partial(
        pl.kernel,
        mesh=_mesh(),
        out_type=jax.ShapeDtypeStruct((NC, N, D), jnp.float32),
        scratch_types=(
            [pltpu.VMEM((CH,), jnp.int32) for _ in range(2 * _NB)]
            + [pltpu.VMEM((CH, D), jnp.float32) for _ in range(_NB)]
            + [pltpu.SemaphoreType.DMA for _ in range(3 * _NB)]
            + [pltpu.VMEM_SHARED((N, D), jnp.float32)]
        ),
    )
    def scat_k(tab_hbm, e_hbm, out_hbm, *refs):
        sidx = refs[0:_NB]
        didx = refs[_NB:2 * _NB]
        rows = refs[2 * _NB:3 * _NB]
        isem = refs[3 * _NB:4 * _NB]
        gsem = refs[4 * _NB:5 * _NB]
        ssem = refs[5 * _NB:6 * _NB]
        acc = refs[6 * _NB]
        cid = lax.axis_index("c")
        sid = lax.axis_index("s")
        z16 = jnp.zeros((16,), jnp.float32)

        def zrow(r, c):
            for k in range(D // 16):
                rows[0][r, pl.ds(k * 16, 16)] = z16
            return c

        lax.fori_loop(0, CH, zrow, 0)
        base_r = sid * _LEN_LO + 8 * jnp.maximum(sid - _SPLIT, 0)

        def zfill(nrows):
            nfull, rem = nrows // CH, nrows % CH
            for c in range(nfull):
                pltpu.sync_copy(rows[0], acc.at[pl.ds(base_r + c * CH, CH)])
            if rem:
                pltpu.sync_copy(rows[0].at[pl.ds(0, rem)],
                                acc.at[pl.ds(base_r + nfull * CH, rem)])

        @pl.when(sid < _SPLIT)
        def _():
            zfill(_LEN_LO)

        @pl.when(sid >= _SPLIT)
        def _():
            zfill(_LEN_LO + 8)

        plsc.subcore_barrier()

        def pipe(nj, base):
            def sslc(j):
                return e_hbm.at[pl.ds((base + j) * CH, CH)]

            def dslc(j):
                return e_hbm.at[pl.ds(ep + (base + j) * CH, CH)]

            def idx_start(s, j):
                pltpu.async_copy(sslc(j), sidx[s], isem[s])
                pltpu.async_copy(dslc(j), didx[s], isem[s])

            def idx_wait(s, j):
                pltpu.make_async_copy(sslc(j), sidx[s], isem[s]).wait()
                pltpu.make_async_copy(dslc(j), didx[s], isem[s]).wait()

            def gat_start(s):
                pltpu.async_copy(tab_hbm.at[sidx[s]], rows[s], gsem[s])

            def gat_wait(s):
                pltpu.make_async_copy(tab_hbm.at[sidx[s]], rows[s],
                                      gsem[s]).wait()

            def sca_start(s):
                pltpu.async_copy(rows[s], acc.at[didx[s]], ssem[s], add=True)

            def sca_wait(s):
                pltpu.make_async_copy(rows[s], acc.at[didx[s]],
                                      ssem[s]).wait()

            idx_start(0, 0)
            for v in range(3):
                idx_wait(v, v)
                gat_start(v)
                idx_start((v + 1) % _NB, min(v + 1, nj - 1))
                if v >= 2:
                    gat_wait((v + 2) % _NB)
                    sca_start((v + 2) % _NB)

            nq = (nj - 4) // _NB

            def steady(i, c):
                v0 = 3 + i * _NB
                for r in range(_NB):
                    b = (3 + r) % _NB
                    v = v0 + r
                    idx_wait(b, v)
                    gat_start(b)
                    sca_wait((b + 1) % _NB)
                    idx_start((b + 1) % _NB, v + 1)
                    gat_wait((b + 2) % _NB)
                    sca_start((b + 2) % _NB)
                return c

            lax.fori_loop(0, nq, steady, 0)

            for v in range(3 + _NB * nq, nj - 1):
                b = v % _NB
                idx_wait(b, v)
                gat_start(b)
                sca_wait((b + 1) % _NB)
                idx_start((b + 1) % _NB, v + 1)
                gat_wait((b + 2) % _NB)
                sca_start((b + 2) % _NB)

            bl = (nj - 1) % _NB
            idx_wait(bl, nj - 1)
            gat_start(bl)
            sca_wait((bl + 1) % _NB)
            for v in (nj - 3, nj - 2, nj - 1):
                s = v % _NB
                gat_wait(s)
                sca_start(s)
                sca_wait(s)

        @pl.when(cid == 0)
        def _():
            pipe(j0, sid * j0)

        @pl.when(jnp.logical_and(cid == 1, sid < NS - 1))
        def _():
            pipe(j1, NS * j0 + sid * j1)

        @pl.when(jnp.logical_and(cid == 1, sid == NS - 1))
        def _():
            pipe(j1t, NS * j0 + (NS - 1) * j1)

        plsc.subcore_barrier()

        @pl.when(sid < _SPLIT)
        def _():
            pltpu.sync_copy(acc.at[pl.ds(base_r, _LEN_LO)],
                            out_hbm.at[cid, pl.ds(base_r, _LEN_LO)])

        @pl.when(sid >= _SPLIT)
        def _():
            pltpu.sync_copy(acc.at[pl.ds(base_r, _LEN_LO + 8)],
                            out_hbm.at[cid, pl.ds(base_r, _LEN_LO + 8)])

    return scat_k(table, eflat)


_BR = 1000


def _tc_scale_matmul(x, c0, c1, base, w1):
    def body(x_ref, c0_ref, c1_ref, base_ref, w_ref, g_ref, dinv_ref):
        dinv = lax.rsqrt(c0_ref[...] + c1_ref[...] + base_ref[...])
        g_ref[...] = jnp.dot(x_ref[...] * dinv, w_ref[...],
                             preferred_element_type=jnp.float32)
        dinv_ref[...] = dinv

    n = x.shape[0]
    return pl.pallas_call(
        body,
        grid=(n // _BR,),
        in_specs=[
            pl.BlockSpec((_BR, D), lambda i: (i, 0)),
            pl.BlockSpec((_BR, 1), lambda i: (i, 0)),
            pl.BlockSpec((_BR, 1), lambda i: (i, 0)),
            pl.BlockSpec((_BR, 1), lambda i: (i, 0)),
            pl.BlockSpec((D, D), lambda i: (0, 0)),
        ],
        out_specs=[
            pl.BlockSpec((_BR, D), lambda i: (i, 0)),
            pl.BlockSpec((_BR, 1), lambda i: (i, 0)),
        ],
        out_shape=[
            jax.ShapeDtypeStruct((n, D), jnp.float32),
            jax.ShapeDtypeStruct((n, 1), jnp.float32),
        ],
    )(x, c0, c1, base, w1)


def _tc_mid(p, g1, dinv, w, b1, w2):
    def body(p0_ref, p1_ref, g_ref, dinv_ref, w_ref, b_ref, w2_ref, o_ref):
        t = dinv_ref[...] * (p0_ref[0] + p1_ref[0]
                             + w_ref[...] * g_ref[...]) + b_ref[...]
        r = jnp.maximum(t, 0.0)
        o_ref[...] = dinv_ref[...] * jnp.dot(r, w2_ref[...],
                                             preferred_element_type=jnp.float32)

    n = g1.shape[0]
    return pl.pallas_call(
        body,
        grid=(n // _BR,),
        in_specs=[
            pl.BlockSpec((1, _BR, D), lambda i: (0, i, 0)),
            pl.BlockSpec((1, _BR, D), lambda i: (1, i, 0)),
            pl.BlockSpec((_BR, D), lambda i: (i, 0)),
            pl.BlockSpec((_BR, 1), lambda i: (i, 0)),
            pl.BlockSpec((_BR, 1), lambda i: (i, 0)),
            pl.BlockSpec((1, D), lambda i: (0, 0)),
            pl.BlockSpec((D, D), lambda i: (0, 0)),
        ],
        out_specs=pl.BlockSpec((_BR, D), lambda i: (i, 0)),
        out_shape=jax.ShapeDtypeStruct((n, D), jnp.float32),
    )(p, p, g1, dinv, w, b1, w2)


def _tc_final(q, g2, dinv, w, b2):
    def body(q0_ref, q1_ref, g_ref, dinv_ref, w_ref, b_ref, o_ref):
        o_ref[...] = dinv_ref[...] * (q0_ref[0] + q1_ref[0]
                                      + w_ref[...] * g_ref[...]) + b_ref[...]

    n = g2.shape[0]
    return pl.pallas_call(
        body,
        grid=(n // _BR,),
        in_specs=[
            pl.BlockSpec((1, _BR, D), lambda i: (0, i, 0)),
            pl.BlockSpec((1, _BR, D), lambda i: (1, i, 0)),
            pl.BlockSpec((_BR, D), lambda i: (i, 0)),
            pl.BlockSpec((_BR, 1), lambda i: (i, 0)),
            pl.BlockSpec((_BR, 1), lambda i: (i, 0)),
            pl.BlockSpec((1, D), lambda i: (0, 0)),
        ],
        out_specs=pl.BlockSpec((_BR, D), lambda i: (i, 0)),
        out_shape=jax.ShapeDtypeStruct((n, D), jnp.float32),
    )(q, q, g2, dinv, w, b2)


def kernel(x, edge_index, W1, b1, W2, b2):
    ei = edge_index.astype(jnp.int32)
    e = ei.shape[1]

    npad = (-e) % CHD
    if npad:
        ei = jnp.concatenate(
            [ei, jnp.zeros((2, npad), jnp.int32)], axis=1)
    ep = e + npad

    chunks = ep // CH
    j0 = max(_NB, int(chunks * F0) // NS)
    j0 = min(j0, (chunks - NS * _NB) // NS)
    rem = chunks - NS * j0
    j1 = max(_NB, rem // NS)
    j1t = rem - (NS - 1) * j1
    assert j1t >= _NB

    dchunks = ep // CHD
    njd = dchunks // NW
    njd_last = dchunks - (NW - 1) * njd
    assert njd >= 4

    eflat = ei.reshape(2 * ep)

    counts = _sc_degree(eflat, ep, njd, njd_last)
    c0 = counts[0, :N].reshape(N, 1)
    c1 = counts[1, :N].reshape(N, 1)

    base = jnp.ones((N, 1), jnp.float32).at[0, 0].add(-float(npad))
    w = jnp.ones((N, 1), jnp.float32).at[0, 0].add(-float(npad))

    g1, dinv = _tc_scale_matmul(x, c0, c1, base, W1)
    p = _sc_scatter(g1, eflat, ep, j0, j1, j1t)
    g2 = _tc_mid(p, g1, dinv, w, b1.reshape(1, D), W2)
    q = _sc_scatter(g2, eflat, ep, j0, j1, j1t)
    return _tc_final(q, g2, dinv, w, b2.reshape(1, D))


# speedup vs baseline: 3.8642x; 1.0414x over previous
import functools

import jax
import jax.numpy as jnp
from jax import lax
from jax.experimental import pallas as pl
from jax.experimental.pallas import tpu as pltpu
from jax.experimental.pallas import tpu_sc as plsc

N = 10000
D = 128
NC = 2
NS = 16
NW = NC * NS
CH = 64
CHD = 128
F0 = 0.52

COUNT_PAD = 10240
CSEG = COUNT_PAD // NS

_BLK8 = N // 8
_LEN_LO = (_BLK8 // NS) * 8
_EXTRA = _BLK8 % NS
_SPLIT = NS - _EXTRA

_NB = 4


def _mesh():
    return plsc.VectorSubcoreMesh(core_axis_name="c", subcore_axis_name="s")


def _sc_degree(eflat, ep, njd, njd_last):
    @functools.partial(
        pl.kernel,
        mesh=_mesh(),
        out_type=jax.ShapeDtypeStruct((NC, COUNT_PAD), jnp.float32),
        scratch_types=(
            [pltpu.VMEM((CHD,), jnp.float32)]
            + [pltpu.VMEM((CHD,), jnp.int32) for _ in range(3)]
            + [pltpu.SemaphoreType.DMA for _ in range(6)]
            + [pltpu.VMEM((CSEG,), jnp.float32),
               pltpu.VMEM_SHARED((COUNT_PAD,), jnp.float32)]
        ),
    )
    def deg_k(dst_hbm, out_hbm, ones_v, *refs):
        idx = refs[0:3]
        isem = refs[3:6]
        ssem = refs[6:9]
        zrow_v, counts = refs[9], refs[10]
        cid = lax.axis_index("c")
        sid = lax.axis_index("s")
        wid = sid * NC + cid
        z16 = jnp.zeros((16,), jnp.float32)
        o16 = jnp.ones((16,), jnp.float32)
        for k in range(CHD // 16):
            ones_v[pl.ds(k * 16, 16)] = o16

        def zb(i, c):
            zrow_v[pl.ds(i * 16, 16)] = z16
            return c

        lax.fori_loop(0, CSEG // 16, zb, 0)
        pltpu.sync_copy(zrow_v, counts.at[pl.ds(sid * CSEG, CSEG)])
        plsc.subcore_barrier()

        def dpipe(nj, base):
            def dslc(j):
                return dst_hbm.at[pl.ds(ep + (base + j) * CHD, CHD)]

            def idx_start(b, j):
                pltpu.async_copy(dslc(j), idx[b], isem[b])

            def visit(b, j, warm, prefetch):
                pltpu.make_async_copy(dslc(j), idx[b], isem[b]).wait()
                if warm:
                    pltpu.make_async_copy(ones_v, counts.at[idx[(b + 1) % 3]],
                                          ssem[(b + 1) % 3]).wait()
                if prefetch:
                    idx_start((b + 1) % 3, j + 1)
                pltpu.async_copy(ones_v, counts.at[idx[b]], ssem[b], add=True)

            idx_start(0, 0)
            visit(0, 0, False, True)
            visit(1, 1, False, True)

            nq = (nj - 3) // 3

            def steady(i, c):
                for r in range(3):
                    visit((2 + r) % 3, 2 + i * 3 + r, True, True)
                return c

            lax.fori_loop(0, nq, steady, 0)
            for v in range(2 + 3 * nq, nj):
                visit(v % 3, v, True, v + 1 < nj)
            for s in ((nj - 2) % 3, (nj - 1) % 3):
                pltpu.make_async_copy(ones_v, counts.at[idx[s]],
                                      ssem[s]).wait()

        @pl.when(wid < NW - 1)
        def _():
            dpipe(njd, wid * njd)

        @pl.when(wid == NW - 1)
        def _():
            dpipe(njd_last, (NW - 1) * njd)

        plsc.subcore_barrier()
        pltpu.sync_copy(counts.at[pl.ds(sid * CSEG, CSEG)],
                        out_hbm.at[cid, pl.ds(sid * CSEG, CSEG)])

    return deg_k(eflat)


def _sc_scatter(table, eflat, ep, j0, j1, j1t):
    for nj in (j0, j1, j1t):
        assert nj >= _NB

    @functools.partial(
        pl.kernel,
        mesh=_mesh(),
        out_type=jax.ShapeDtypeStruct((NC, N, D), jnp.float32),
        scratch_types=(
            [pltpu.VMEM((CH,), jnp.int32) for _ in range(2 * _NB)]
            + [pltpu.VMEM((CH, D), jnp.float32) for _ in range(_NB)]
            + [pltpu.SemaphoreType.DMA for _ in range(3 * _NB)]
            + [pltpu.VMEM_SHARED((N, D), jnp.float32)]
        ),
    )
    def scat_k(tab_hbm, e_hbm, out_hbm, *refs):
        sidx = refs[0:_NB]
        didx = refs[_NB:2 * _NB]
        rows = refs[2 * _NB:3 * _NB]
        isem = refs[3 * _NB:4 * _NB]
        gsem = refs[4 * _NB:5 * _NB]
        ssem = refs[5 * _NB:6 * _NB]
        acc = refs[6 * _NB]
        cid = lax.axis_index("c")
        sid = lax.axis_index("s")
        z16 = jnp.zeros((16,), jnp.float32)

        def zrow(r, c):
            for k in range(D // 16):
                rows[0][r, pl.ds(k * 16, 16)] = z16
            return c

        lax.fori_loop(0, CH, zrow, 0)
        base_r = sid * _LEN_LO + 8 * jnp.maximum(sid - _SPLIT, 0)

        def zfill(nrows):
            nfull, rem = nrows // CH, nrows % CH
            for c in range(nfull):
                pltpu.sync_copy(rows[0], acc.at[pl.ds(base_r + c * CH, CH)])
            if rem:
                pltpu.sync_copy(rows[0].at[pl.ds(0, rem)],
                                acc.at[pl.ds(base_r + nfull * CH, rem)])

        @pl.when(sid < _SPLIT)
        def _():
            zfill(_LEN_LO)

        @pl.when(sid >= _SPLIT)
        def _():
            zfill(_LEN_LO + 8)

        plsc.subcore_barrier()

        def pipe(nj, base):
            def sslc(j):
                return e_hbm.at[pl.ds((base + j) * CH, CH)]

            def dslc(j):
                return e_hbm.at[pl.ds(ep + (base + j) * CH, CH)]

            def idx_start(s, j):
                pltpu.async_copy(sslc(j), sidx[s], isem[s])
                pltpu.async_copy(dslc(j), didx[s], isem[s])

            def idx_wait(s, j):
                pltpu.make_async_copy(sslc(j), sidx[s], isem[s]).wait()
                pltpu.make_async_copy(dslc(j), didx[s], isem[s]).wait()

            def gat_start(s):
                pltpu.async_copy(tab_hbm.at[sidx[s]], rows[s], gsem[s])

            def gat_wait(s):
                pltpu.make_async_copy(tab_hbm.at[sidx[s]], rows[s],
                                      gsem[s]).wait()

            def sca_start(s):
                pltpu.async_copy(rows[s], acc.at[didx[s]], ssem[s], add=True)

            def sca_wait(s):
                pltpu.make_async_copy(rows[s], acc.at[didx[s]],
                                      ssem[s]).wait()

            idx_start(0, 0)
            for v in range(3):
                idx_wait(v, v)
                gat_start(v)
                idx_start((v + 1) % _NB, min(v + 1, nj - 1))
                if v >= 2:
                    gat_wait((v + 2) % _NB)
                    sca_start((v + 2) % _NB)

            nq = (nj - 4) // _NB

            def steady(i, c):
                v0 = 3 + i * _NB
                for r in range(_NB):
                    b = (3 + r) % _NB
                    v = v0 + r
                    idx_wait(b, v)
                    gat_start(b)
                    sca_wait((b + 1) % _NB)
                    idx_start((b + 1) % _NB, v + 1)
                    gat_wait((b + 2) % _NB)
                    sca_start((b + 2) % _NB)
                return c

            lax.fori_loop(0, nq, steady, 0)

            for v in range(3 + _NB * nq, nj - 1):
                b = v % _NB
                idx_wait(b, v)
                gat_start(b)
                sca_wait((b + 1) % _NB)
                idx_start((b + 1) % _NB, v + 1)
                gat_wait((b + 2) % _NB)
                sca_start((b + 2) % _NB)

            bl = (nj - 1) % _NB
            idx_wait(bl, nj - 1)
            gat_start(bl)
            sca_wait((bl + 1) % _NB)
            for v in (nj - 3, nj - 2, nj - 1):
                s = v % _NB
                gat_wait(s)
                sca_start(s)
                sca_wait(s)

        @pl.when(cid == 0)
        def _():
            pipe(j0, sid * j0)

        @pl.when(jnp.logical_and(cid == 1, sid < NS - 1))
        def _():
            pipe(j1, NS * j0 + sid * j1)

        @pl.when(jnp.logical_and(cid == 1, sid == NS - 1))
        def _():
            pipe(j1t, NS * j0 + (NS - 1) * j1)

        plsc.subcore_barrier()

        @pl.when(sid < _SPLIT)
        def _():
            pltpu.sync_copy(acc.at[pl.ds(base_r, _LEN_LO)],
                            out_hbm.at[cid, pl.ds(base_r, _LEN_LO)])

        @pl.when(sid >= _SPLIT)
        def _():
            pltpu.sync_copy(acc.at[pl.ds(base_r, _LEN_LO + 8)],
                            out_hbm.at[cid, pl.ds(base_r, _LEN_LO + 8)])

    return scat_k(table, eflat)


_BR = 1000


def _tc_scale_matmul(x, c0, c1, base, w1):
    def body(x_ref, c0_ref, c1_ref, base_ref, w_ref, g_ref, dinv_ref):
        dinv = lax.rsqrt(c0_ref[...] + c1_ref[...] + base_ref[...])
        g_ref[...] = jnp.dot(x_ref[...] * dinv, w_ref[...],
                             preferred_element_type=jnp.float32)
        dinv_ref[...] = dinv

    n = x.shape[0]
    return pl.pallas_call(
        body,
        grid=(n // _BR,),
        in_specs=[
            pl.BlockSpec((_BR, D), lambda i: (i, 0)),
            pl.BlockSpec((_BR, 1), lambda i: (i, 0)),
            pl.BlockSpec((_BR, 1), lambda i: (i, 0)),
            pl.BlockSpec((_BR, 1), lambda i: (i, 0)),
            pl.BlockSpec((D, D), lambda i: (0, 0)),
        ],
        out_specs=[
            pl.BlockSpec((_BR, D), lambda i: (i, 0)),
            pl.BlockSpec((_BR, 1), lambda i: (i, 0)),
        ],
        out_shape=[
            jax.ShapeDtypeStruct((n, D), jnp.float32),
            jax.ShapeDtypeStruct((n, 1), jnp.float32),
        ],
    )(x, c0, c1, base, w1)


def _tc_mid(p, g1, dinv, w, b1, w2):
    def body(p0_ref, p1_ref, g_ref, dinv_ref, w_ref, b_ref, w2_ref, o_ref):
        t = dinv_ref[...] * (p0_ref[0] + p1_ref[0]
                             + w_ref[...] * g_ref[...]) + b_ref[...]
        r = jnp.maximum(t, 0.0)
        o_ref[...] = dinv_ref[...] * jnp.dot(r, w2_ref[...],
                                             preferred_element_type=jnp.float32)

    n = g1.shape[0]
    return pl.pallas_call(
        body,
        grid=(n // _BR,),
        in_specs=[
            pl.BlockSpec((1, _BR, D), lambda i: (0, i, 0)),
            pl.BlockSpec((1, _BR, D), lambda i: (1, i, 0)),
            pl.BlockSpec((_BR, D), lambda i: (i, 0)),
            pl.BlockSpec((_BR, 1), lambda i: (i, 0)),
            pl.BlockSpec((_BR, 1), lambda i: (i, 0)),
            pl.BlockSpec((1, D), lambda i: (0, 0)),
            pl.BlockSpec((D, D), lambda i: (0, 0)),
        ],
        out_specs=pl.BlockSpec((_BR, D), lambda i: (i, 0)),
        out_shape=jax.ShapeDtypeStruct((n, D), jnp.float32),
    )(p, p, g1, dinv, w, b1, w2)


def _tc_final(q, g2, dinv, w, b2):
    def body(q0_ref, q1_ref, g_ref, dinv_ref, w_ref, b_ref, o_ref):
        o_ref[...] = dinv_ref[...] * (q0_ref[0] + q1_ref[0]
                                      + w_ref[...] * g_ref[...]) + b_ref[...]

    n = g2.shape[0]
    return pl.pallas_call(
        body,
        grid=(n // _BR,),
        in_specs=[
            pl.BlockSpec((1, _BR, D), lambda i: (0, i, 0)),
            pl.BlockSpec((1, _BR, D), lambda i: (1, i, 0)),
            pl.BlockSpec((_BR, D), lambda i: (i, 0)),
            pl.BlockSpec((_BR, 1), lambda i: (i, 0)),
            pl.BlockSpec((_BR, 1), lambda i: (i, 0)),
            pl.BlockSpec((1, D), lambda i: (0, 0)),
        ],
        out_specs=pl.BlockSpec((_BR, D), lambda i: (i, 0)),
        out_shape=jax.ShapeDtypeStruct((n, D), jnp.float32),
    )(q, q, g2, dinv, w, b2)


def kernel(x, edge_index, W1, b1, W2, b2):
    ei = edge_index.astype(jnp.int32)
    e = ei.shape[1]

    npad = (-e) % CHD
    if npad:
        ei = jnp.concatenate(
            [ei, jnp.zeros((2, npad), jnp.int32)], axis=1)
    ep = e + npad

    chunks = ep // CH
    j0 = max(_NB, int(chunks * F0) // NS)
    j0 = min(j0, (chunks - NS * _NB) // NS)
    rem = chunks - NS * j0
    j1 = max(_NB, rem // NS)
    j1t = rem - (NS - 1) * j1
    assert j1t >= _NB

    dchunks = ep // CHD
    njd = dchunks // NW
    njd_last = dchunks - (NW - 1) * njd
    assert njd >= 4

    eflat = ei.reshape(2 * ep)

    counts = _sc_degree(eflat, ep, njd, njd_last)
    c0 = counts[0, :N].reshape(N, 1)
    c1 = counts[1, :N].reshape(N, 1)

    base = jnp.ones((N, 1), jnp.float32).at[0, 0].add(-float(npad))
    w = jnp.ones((N, 1), jnp.float32).at[0, 0].add(-float(npad))

    g1, dinv = _tc_scale_matmul(x, c0, c1, base, W1)
    p = _sc_scatter(g1, eflat, ep, j0, j1, j1t)
    g2 = _tc_mid(p, g1, dinv, w, b1.reshape(1, D), W2)
    q = _sc_scatter(g2, eflat, ep, j0, j1, j1t)
    return _tc_final(q, g2, dinv, w, b2.reshape(1, D))


# speedup vs baseline: 3.8683x; 1.0011x over previous
import functools

import jax
import jax.numpy as jnp
from jax import lax
from jax.experimental import pallas as pl
from jax.experimental.pallas import tpu as pltpu
from jax.experimental.pallas import tpu_sc as plsc

N = 10000
D = 128
NC = 2
NS = 16
NW = NC * NS
CH = 64
CHD = 128
F0 = 0.48

COUNT_PAD = 10240
CSEG = COUNT_PAD // NS

_BLK8 = N // 8
_LEN_LO = (_BLK8 // NS) * 8
_EXTRA = _BLK8 % NS
_SPLIT = NS - _EXTRA

_NB = 4


def _mesh():
    return plsc.VectorSubcoreMesh(core_axis_name="c", subcore_axis_name="s")


def _sc_degree(eflat, ep, njd, njd_last):
    @functools.partial(
        pl.kernel,
        mesh=_mesh(),
        out_type=jax.ShapeDtypeStruct((NC, COUNT_PAD), jnp.float32),
        scratch_types=(
            [pltpu.VMEM((CHD,), jnp.float32)]
            + [pltpu.VMEM((CHD,), jnp.int32) for _ in range(3)]
            + [pltpu.SemaphoreType.DMA for _ in range(6)]
            + [pltpu.VMEM((CSEG,), jnp.float32),
               pltpu.VMEM_SHARED((COUNT_PAD,), jnp.float32)]
        ),
    )
    def deg_k(dst_hbm, out_hbm, ones_v, *refs):
        idx = refs[0:3]
        isem = refs[3:6]
        ssem = refs[6:9]
        zrow_v, counts = refs[9], refs[10]
        cid = lax.axis_index("c")
        sid = lax.axis_index("s")
        wid = sid * NC + cid
        z16 = jnp.zeros((16,), jnp.float32)
        o16 = jnp.ones((16,), jnp.float32)
        for k in range(CHD // 16):
            ones_v[pl.ds(k * 16, 16)] = o16

        def zb(i, c):
            zrow_v[pl.ds(i * 16, 16)] = z16
            return c

        lax.fori_loop(0, CSEG // 16, zb, 0)
        pltpu.sync_copy(zrow_v, counts.at[pl.ds(sid * CSEG, CSEG)])
        plsc.subcore_barrier()

        def dpipe(nj, base):
            def dslc(j):
                return dst_hbm.at[pl.ds(ep + (base + j) * CHD, CHD)]

            def idx_start(b, j):
                pltpu.async_copy(dslc(j), idx[b], isem[b])

            def visit(b, j, warm, prefetch):
                pltpu.make_async_copy(dslc(j), idx[b], isem[b]).wait()
                if warm:
                    pltpu.make_async_copy(ones_v, counts.at[idx[(b + 1) % 3]],
                                          ssem[(b + 1) % 3]).wait()
                if prefetch:
                    idx_start((b + 1) % 3, j + 1)
                pltpu.async_copy(ones_v, counts.at[idx[b]], ssem[b], add=True)

            idx_start(0, 0)
            visit(0, 0, False, True)
            visit(1, 1, False, True)

            nq = (nj - 3) // 3

            def steady(i, c):
                for r in range(3):
                    visit((2 + r) % 3, 2 + i * 3 + r, True, True)
                return c

            lax.fori_loop(0, nq, steady, 0)
            for v in range(2 + 3 * nq, nj):
                visit(v % 3, v, True, v + 1 < nj)
            for s in ((nj - 2) % 3, (nj - 1) % 3):
                pltpu.make_async_copy(ones_v, counts.at[idx[s]],
                                      ssem[s]).wait()

        @pl.when(wid < NW - 1)
        def _():
            dpipe(njd, wid * njd)

        @pl.when(wid == NW - 1)
        def _():
            dpipe(njd_last, (NW - 1) * njd)

        plsc.subcore_barrier()
        pltpu.sync_copy(counts.at[pl.ds(sid * CSEG, CSEG)],
                        out_hbm.at[cid, pl.ds(sid * CSEG, CSEG)])

    return deg_k(eflat)


def _sc_scatter(table, eflat, ep, j0, j1, j1t):
    for nj in (j0, j1, j1t):
        assert nj >= _NB

    @functools.partial(
        pl.kernel,
        mesh=_mesh(),
        out_type=jax.ShapeDtypeStruct((NC, N, D), jnp.float32),
        scratch_types=(
            [pltpu.VMEM((CH,), jnp.int32) for _ in range(2 * _NB)]
            + [pltpu.VMEM((CH, D), jnp.float32) for _ in range(_NB)]
            + [pltpu.SemaphoreType.DMA for _ in range(3 * _NB)]
            + [pltpu.VMEM_SHARED((N, D), jnp.float32)]
        ),
    )
    def scat_k(tab_hbm, e_hbm, out_hbm, *refs):
        sidx = refs[0:_NB]
        didx = refs[_NB:2 * _NB]
        rows = refs[2 * _NB:3 * _NB]
        isem = refs[3 * _NB:4 * _NB]
        gsem = refs[4 * _NB:5 * _NB]
        ssem = refs[5 * _NB:6 * _NB]
        acc = refs[6 * _NB]
        cid = lax.axis_index("c")
        sid = lax.axis_index("s")
        z16 = jnp.zeros((16,), jnp.float32)

        def zrow(r, c):
            for k in range(D // 16):
                rows[0][r, pl.ds(k * 16, 16)] = z16
            return c

        lax.fori_loop(0, CH, zrow, 0)
        base_r = sid * _LEN_LO + 8 * jnp.maximum(sid - _SPLIT, 0)

        def zfill(nrows):
            nfull, rem = nrows // CH, nrows % CH
            for c in range(nfull):
                pltpu.sync_copy(rows[0], acc.at[pl.ds(base_r + c * CH, CH)])
            if rem:
                pltpu.sync_copy(rows[0].at[pl.ds(0, rem)],
                                acc.at[pl.ds(base_r + nfull * CH, rem)])

        @pl.when(sid < _SPLIT)
        def _():
            zfill(_LEN_LO)

        @pl.when(sid >= _SPLIT)
        def _():
            zfill(_LEN_LO + 8)

        plsc.subcore_barrier()

        def pipe(nj, base):
            def sslc(j):
                return e_hbm.at[pl.ds((base + j) * CH, CH)]

            def dslc(j):
                return e_hbm.at[pl.ds(ep + (base + j) * CH, CH)]

            def idx_start(s, j):
                pltpu.async_copy(sslc(j), sidx[s], isem[s])
                pltpu.async_copy(dslc(j), didx[s], isem[s])

            def idx_wait(s, j):
                pltpu.make_async_copy(sslc(j), sidx[s], isem[s]).wait()
                pltpu.make_async_copy(dslc(j), didx[s], isem[s]).wait()

            def gat_start(s):
                pltpu.async_copy(tab_hbm.at[sidx[s]], rows[s], gsem[s])

            def gat_wait(s):
                pltpu.make_async_copy(tab_hbm.at[sidx[s]], rows[s],
                                      gsem[s]).wait()

            def sca_start(s):
                pltpu.async_copy(rows[s], acc.at[didx[s]], ssem[s], add=True)

            def sca_wait(s):
                pltpu.make_async_copy(rows[s], acc.at[didx[s]],
                                      ssem[s]).wait()

            idx_start(0, 0)
            for v in range(3):
                idx_wait(v, v)
                gat_start(v)
                idx_start((v + 1) % _NB, min(v + 1, nj - 1))
                if v >= 2:
                    gat_wait((v + 2) % _NB)
                    sca_start((v + 2) % _NB)

            nq = (nj - 4) // _NB

            def steady(i, c):
                v0 = 3 + i * _NB
                for r in range(_NB):
                    b = (3 + r) % _NB
                    v = v0 + r
                    idx_wait(b, v)
                    gat_start(b)
                    sca_wait((b + 1) % _NB)
                    idx_start((b + 1) % _NB, v + 1)
                    gat_wait((b + 2) % _NB)
                    sca_start((b + 2) % _NB)
                return c

            lax.fori_loop(0, nq, steady, 0)

            for v in range(3 + _NB * nq, nj - 1):
                b = v % _NB
                idx_wait(b, v)
                gat_start(b)
                sca_wait((b + 1) % _NB)
                idx_start((b + 1) % _NB, v + 1)
                gat_wait((b + 2) % _NB)
                sca_start((b + 2) % _NB)

            bl = (nj - 1) % _NB
            idx_wait(bl, nj - 1)
            gat_start(bl)
            sca_wait((bl + 1) % _NB)
            for v in (nj - 3, nj - 2, nj - 1):
                s = v % _NB
                gat_wait(s)
                sca_start(s)
                sca_wait(s)

        @pl.when(cid == 0)
        def _():
            pipe(j0, sid * j0)

        @pl.when(jnp.logical_and(cid == 1, sid < NS - 1))
        def _():
            pipe(j1, NS * j0 + sid * j1)

        @pl.when(jnp.logical_and(cid == 1, sid == NS - 1))
        def _():
            pipe(j1t, NS * j0 + (NS - 1) * j1)

        plsc.subcore_barrier()

        @pl.when(sid < _SPLIT)
        def _():
            pltpu.sync_copy(acc.at[pl.ds(base_r, _LEN_LO)],
                            out_hbm.at[cid, pl.ds(base_r, _LEN_LO)])

        @pl.when(sid >= _SPLIT)
        def _():
            pltpu.sync_copy(acc.at[pl.ds(base_r, _LEN_LO + 8)],
                            out_hbm.at[cid, pl.ds(base_r, _LEN_LO + 8)])

    return scat_k(table, eflat)


_BR = 1000


def _tc_scale_matmul(x, c0, c1, base, w1):
    def body(x_ref, c0_ref, c1_ref, base_ref, w_ref, g_ref, dinv_ref):
        dinv = lax.rsqrt(c0_ref[...] + c1_ref[...] + base_ref[...])
        g_ref[...] = jnp.dot(x_ref[...] * dinv, w_ref[...],
                             preferred_element_type=jnp.float32)
        dinv_ref[...] = dinv

    n = x.shape[0]
    return pl.pallas_call(
        body,
        grid=(n // _BR,),
        in_specs=[
            pl.BlockSpec((_BR, D), lambda i: (i, 0)),
            pl.BlockSpec((_BR, 1), lambda i: (i, 0)),
            pl.BlockSpec((_BR, 1), lambda i: (i, 0)),
            pl.BlockSpec((_BR, 1), lambda i: (i, 0)),
            pl.BlockSpec((D, D), lambda i: (0, 0)),
        ],
        out_specs=[
            pl.BlockSpec((_BR, D), lambda i: (i, 0)),
            pl.BlockSpec((_BR, 1), lambda i: (i, 0)),
        ],
        out_shape=[
            jax.ShapeDtypeStruct((n, D), jnp.float32),
            jax.ShapeDtypeStruct((n, 1), jnp.float32),
        ],
    )(x, c0, c1, base, w1)


def _tc_mid(p, g1, dinv, w, b1, w2):
    def body(p0_ref, p1_ref, g_ref, dinv_ref, w_ref, b_ref, w2_ref, o_ref):
        t = dinv_ref[...] * (p0_ref[0] + p1_ref[0]
                             + w_ref[...] * g_ref[...]) + b_ref[...]
        r = jnp.maximum(t, 0.0)
        o_ref[...] = dinv_ref[...] * jnp.dot(r, w2_ref[...],
                                             preferred_element_type=jnp.float32)

    n = g1.shape[0]
    return pl.pallas_call(
        body,
        grid=(n // _BR,),
        in_specs=[
            pl.BlockSpec((1, _BR, D), lambda i: (0, i, 0)),
            pl.BlockSpec((1, _BR, D), lambda i: (1, i, 0)),
            pl.BlockSpec((_BR, D), lambda i: (i, 0)),
            pl.BlockSpec((_BR, 1), lambda i: (i, 0)),
            pl.BlockSpec((_BR, 1), lambda i: (i, 0)),
            pl.BlockSpec((1, D), lambda i: (0, 0)),
            pl.BlockSpec((D, D), lambda i: (0, 0)),
        ],
        out_specs=pl.BlockSpec((_BR, D), lambda i: (i, 0)),
        out_shape=jax.ShapeDtypeStruct((n, D), jnp.float32),
    )(p, p, g1, dinv, w, b1, w2)


def _tc_final(q, g2, dinv, w, b2):
    def body(q0_ref, q1_ref, g_ref, dinv_ref, w_ref, b_ref, o_ref):
        o_ref[...] = dinv_ref[...] * (q0_ref[0] + q1_ref[0]
                                      + w_ref[...] * g_ref[...]) + b_ref[...]

    n = g2.shape[0]
    return pl.pallas_call(
        body,
        grid=(n // _BR,),
        in_specs=[
            pl.BlockSpec((1, _BR, D), lambda i: (0, i, 0)),
            pl.BlockSpec((1, _BR, D), lambda i: (1, i, 0)),
            pl.BlockSpec((_BR, D), lambda i: (i, 0)),
            pl.BlockSpec((_BR, 1), lambda i: (i, 0)),
            pl.BlockSpec((_BR, 1), lambda i: (i, 0)),
            pl.BlockSpec((1, D), lambda i: (0, 0)),
        ],
        out_specs=pl.BlockSpec((_BR, D), lambda i: (i, 0)),
        out_shape=jax.ShapeDtypeStruct((n, D), jnp.float32),
    )(q, q, g2, dinv, w, b2)


def kernel(x, edge_index, W1, b1, W2, b2):
    ei = edge_index.astype(jnp.int32)
    e = ei.shape[1]

    npad = (-e) % CHD
    if npad:
        ei = jnp.concatenate(
            [ei, jnp.zeros((2, npad), jnp.int32)], axis=1)
    ep = e + npad

    chunks = ep // CH
    j0 = max(_NB, int(chunks * F0) // NS)
    j0 = min(j0, (chunks - NS * _NB) // NS)
    rem = chunks - NS * j0
    j1 = max(_NB, rem // NS)
    j1t = rem - (NS - 1) * j1
    assert j1t >= _NB

    dchunks = ep // CHD
    njd = dchunks // NW
    njd_last = dchunks - (NW - 1) * njd
    assert njd >= 4

    eflat = ei.reshape(2 * ep)

    counts = _sc_degree(eflat, ep, njd, njd_last)
    c0 = counts[0, :N].reshape(N, 1)
    c1 = counts[1, :N].reshape(N, 1)

    base = jnp.ones((N, 1), jnp.float32).at[0, 0].add(-float(npad))
    w = jnp.ones((N, 1), jnp.float32).at[0, 0].add(-float(npad))

    g1, dinv = _tc_scale_matmul(x, c0, c1, base, W1)
    p = _sc_scatter(g1, eflat, ep, j0, j1, j1t)
    g2 = _tc_mid(p, g1, dinv, w, b1.reshape(1, D), W2)
    q = _sc_scatter(g2, eflat, ep, j0, j1, j1t)
    return _tc_final(q, g2, dinv, w, b2.reshape(1, D))
